# Initial kernel scaffold; baseline (speedup 1.0000x reference)
import functools

import jax
import jax.numpy as jnp
from jax import lax
from jax.experimental import pallas as pl
from jax.experimental.pallas import tpu as pltpu
from jax.experimental.pallas import tpu_sc as plsc

_NC = 2
_NS = 16
_LANES = 16


def _sc_degree(dst, n_pad):
  e = dst.shape[0]
  nw = _NC * _NS
  per = e // nw
  assert per * nw == e and per % _LANES == 0
  mesh = plsc.VectorSubcoreMesh(core_axis_name="c", subcore_axis_name="s")

  @functools.partial(
      pl.kernel,
      out_type=jax.ShapeDtypeStruct((_NC, _NS, n_pad), jnp.float32),
      mesh=mesh,
      scratch_types=[
          pltpu.VMEM((n_pad,), jnp.float32),
          pltpu.VMEM((per,), jnp.int32),
      ],
  )
  def k(dst_hbm, out_hbm, hist_v, idx_v):
    c = lax.axis_index("c")
    s = lax.axis_index("s")
    w = c * _NS + s

    @pl.loop(0, n_pad // _LANES)
    def _(i):
      hist_v[pl.ds(i * _LANES, _LANES)] = jnp.zeros((_LANES,), jnp.float32)

    pltpu.sync_copy(dst_hbm.at[pl.ds(w * per, per)], idx_v)
    ones = jnp.ones((_LANES,), jnp.float32)

    @pl.loop(0, per // _LANES)
    def _(i):
      idx = idx_v[pl.ds(i * _LANES, _LANES)]
      plsc.addupdate_scatter(hist_v, [idx], ones)

    pltpu.sync_copy(hist_v, out_hbm.at[c, s])

  return k(dst)


def _sc_scatter(y, src, dst):
  n, d = y.shape
  e = src.shape[0]
  nw = _NC * _NS
  per = e // nw
  k_ch = 80
  nch = per // k_ch
  assert per * nw == e and nch * k_ch == per
  rows_per_tile = n // _NS
  zr = 125
  nz = rows_per_tile // zr
  assert zr * nz == rows_per_tile
  mesh = plsc.VectorSubcoreMesh(core_axis_name="c", subcore_axis_name="s")

  @functools.partial(
      pl.kernel,
      out_type=jax.ShapeDtypeStruct((_NC, n, d), jnp.float32),
      mesh=mesh,
      scratch_types=[
          pltpu.VMEM((k_ch,), jnp.int32),
          pltpu.VMEM((k_ch,), jnp.int32),
          pltpu.VMEM((k_ch, d), jnp.float32),
          pltpu.VMEM((zr, d), jnp.float32),
          pltpu.VMEM_SHARED((n, d), jnp.float32),
      ],
  )
  def k(y_hbm, src_hbm, dst_hbm, out_hbm, src_v, dst_v, rows_v, zero_v, acc_sh):
    c = lax.axis_index("c")
    s = lax.axis_index("s")

    @pl.loop(0, zr)
    def _(i):
      for j in range(d // _LANES):
        zero_v[i, pl.ds(j * _LANES, _LANES)] = jnp.zeros((_LANES,), jnp.float32)

    for t in range(nz):
      pltpu.sync_copy(zero_v, acc_sh.at[pl.ds(s * rows_per_tile + t * zr, zr)])
    plsc.subcore_barrier()

    base0 = (c * _NS + s) * per

    @pl.loop(0, nch)
    def _(i):
      base = base0 + i * k_ch
      pltpu.sync_copy(src_hbm.at[pl.ds(base, k_ch)], src_v)
      pltpu.sync_copy(dst_hbm.at[pl.ds(base, k_ch)], dst_v)
      pltpu.sync_copy(y_hbm.at[src_v], rows_v)
      pltpu.sync_copy(rows_v, acc_sh.at[dst_v], add=True)

    plsc.subcore_barrier()
    pltpu.sync_copy(
        acc_sh.at[pl.ds(s * rows_per_tile, rows_per_tile)],
        out_hbm.at[c, pl.ds(s * rows_per_tile, rows_per_tile)],
    )

  return k(y, src, dst)


def _tc_first(deg_t, x, w1):
  n, d_in = x.shape
  n_pad = deg_t.shape[0]
  h = w1.shape[1]

  def body(deg_ref, x_ref, w_ref, dis_ref, y_ref):
    deg = jnp.sum(deg_ref[...], axis=1, keepdims=True) + 1.0
    dis = lax.rsqrt(deg)
    dis_ref[...] = dis
    mm = jnp.dot(x_ref[...], w_ref[...], preferred_element_type=jnp.float32)
    y_ref[...] = mm * dis[:n]

  return pl.pallas_call(
      body,
      out_shape=(
          jax.ShapeDtypeStruct((n_pad, 1), jnp.float32),
          jax.ShapeDtypeStruct((n, h), jnp.float32),
      ),
  )(deg_t, x, w1)


def _tc_layer(parts, y, dis, b, w_next):
  n, d = y.shape

  def body(p_ref, y_ref, d_ref, b_ref, w_ref, o_ref):
    t = (p_ref[0] + p_ref[1] + y_ref[...]) * d_ref[...] + b_ref[...]
    z = jnp.maximum(t, 0.0)
    o_ref[...] = (
        jnp.dot(z, w_ref[...], preferred_element_type=jnp.float32) * d_ref[...]
    )

  return pl.pallas_call(
      body,
      out_shape=jax.ShapeDtypeStruct((n, d), jnp.float32),
  )(parts, y, dis, b, w_next)


def _tc_final(parts, y, dis, b, batch_row, batch_col, gstats,
              a_mean, a_max, a_sum, a_st, mb1, m2, mb2, num_graphs):
  n, d = y.shape
  g = num_graphs

  def body(p_ref, y_ref, d_ref, b_ref, br_ref, bc_ref, gs_ref,
           am_ref, ax_ref, as_ref, ast_ref, mb1_ref, m2_ref, mb2_ref, o_ref):
    t = (p_ref[0] + p_ref[1] + y_ref[...]) * d_ref[...] + b_ref[...]
    h = jnp.maximum(t, 0.0)

    gid = lax.broadcasted_iota(jnp.int32, (g, 1), 0)
    oh = (br_ref[...] == gid).astype(jnp.float32)
    ssum = jnp.dot(oh, h, preferred_element_type=jnp.float32)
    cnt = jnp.sum(oh, axis=1, keepdims=True)
    mean = ssum / jnp.maximum(cnt, 1.0)

    bc = bc_ref[...]
    neg = jnp.float32(-jnp.inf)
    mx_rows = []
    for gg in range(g):
      m = jnp.where(bc == gg, h, neg)
      mx_rows.append(jnp.max(m, axis=0, keepdims=True))
    mx = jnp.concatenate(mx_rows, axis=0)

    zpre = (
        jnp.dot(mean, am_ref[...], preferred_element_type=jnp.float32)
        + jnp.dot(mx, ax_ref[...], preferred_element_type=jnp.float32)
        + jnp.dot(ssum, as_ref[...], preferred_element_type=jnp.float32)
        + mb1_ref[...]
    )
    gs = gs_ref[...]
    for kk in range(gs.shape[1]):
      zpre = zpre + gs[:, kk:kk + 1] * ast_ref[kk:kk + 1, :]
    z = jnp.maximum(zpre, 0.0)
    o_ref[...] = (
        jnp.dot(z, m2_ref[...], preferred_element_type=jnp.float32)
        + mb2_ref[...]
    )

  return pl.pallas_call(
      body,
      out_shape=jax.ShapeDtypeStruct((g, 1), jnp.float32),
  )(parts, y, dis, b, batch_row, batch_col, gstats,
    a_mean, a_max, a_sum, a_st, mb1, m2, mb2)


def kernel(x, edge_index, batch, graph_stats, W1, b1, W2, b2, W3, b3,
           M1, mb1, M2, mb2):
  n, d_in = x.shape
  h = W1.shape[1]
  g = graph_stats.shape[0]
  src = edge_index[0].astype(jnp.int32)
  dst = edge_index[1].astype(jnp.int32)

  n_pad = ((n + 16 * _LANES - 1) // (16 * _LANES)) * (16 * _LANES)
  deg_parts = _sc_degree(dst, n_pad)
  deg_t = deg_parts.reshape(_NC * _NS, n_pad).T

  dis_full, y = _tc_first(deg_t, x, W1)
  dis = dis_full[:n]

  parts = _sc_scatter(y, src, dst)
  y = _tc_layer(parts, y, dis, b1.reshape(1, h), W2)
  parts = _sc_scatter(y, src, dst)
  y = _tc_layer(parts, y, dis, b2.reshape(1, h), W3)
  parts = _sc_scatter(y, src, dst)

  a_mean = M1[:h]
  a_max = M1[h:2 * h]
  a_sum = M1[2 * h:3 * h]
  a_st = M1[3 * h:]
  out = _tc_final(
      parts, y, dis, b3.reshape(1, h),
      batch.astype(jnp.int32).reshape(1, n),
      batch.astype(jnp.int32).reshape(n, 1),
      graph_stats,
      a_mean, a_max, a_sum, a_st,
      mb1.reshape(1, -1), M2, mb2.reshape(1, 1), g,
  )
  return jnp.squeeze(out)


# speedup vs baseline: 11.8269x; 11.8269x over previous
import dataclasses
import functools

import jax
import jax.numpy as jnp
from jax import lax
from jax.experimental import pallas as pl
from jax.experimental.pallas import tpu as pltpu
from jax.experimental.pallas import tpu_sc as plsc

_NC = 2
_NS = 16
_LANES = 16


def _sc_params():
  cp = pltpu.CompilerParams()
  if "needs_layout_passes" in pltpu.CompilerParams.__dataclass_fields__:
    cp = dataclasses.replace(cp, needs_layout_passes=False)
  return cp


def _sc_degree(dst, n_pad):
  e = dst.shape[0]
  nw = _NC * _NS
  per = e // nw
  assert per * nw == e and per % _LANES == 0
  mesh = plsc.VectorSubcoreMesh(core_axis_name="c", subcore_axis_name="s")

  @functools.partial(
      pl.kernel,
      out_type=jax.ShapeDtypeStruct((nw * n_pad,), jnp.float32),
      mesh=mesh,
      scratch_types=[
          pltpu.VMEM((n_pad,), jnp.float32),
          pltpu.VMEM((per,), jnp.int32),
      ],
      compiler_params=_sc_params(),
  )
  def k(dst_hbm, out_hbm, hist_v, idx_v):
    c = lax.axis_index("c")
    s = lax.axis_index("s")
    w = c * _NS + s

    @pl.loop(0, n_pad // _LANES)
    def _(i):
      hist_v[pl.ds(i * _LANES, _LANES)] = jnp.zeros((_LANES,), jnp.float32)

    pltpu.sync_copy(dst_hbm.at[pl.ds(w * per, per)], idx_v)
    ones = jnp.ones((_LANES,), jnp.float32)

    @pl.loop(0, per // _LANES)
    def _(i):
      idx = idx_v[pl.ds(i * _LANES, _LANES)]
      plsc.addupdate_scatter(hist_v, [idx], ones)

    pltpu.sync_copy(hist_v, out_hbm.at[pl.ds(w * n_pad, n_pad)])

  return k(dst)


def _sc_scatter(y, src, dst):
  n, d = y.shape
  e = src.shape[0]
  nw = _NC * _NS
  per = e // nw
  k_ch = 80
  nch = per // k_ch
  assert per * nw == e and nch * k_ch == per
  rpt = (n // _NS) // 8 * 8
  rem = n - rpt * _NS
  assert rem % 8 == 0
  zr = 156
  nz = rpt // zr
  assert zr * nz == rpt and rem <= zr
  mesh = plsc.VectorSubcoreMesh(core_axis_name="c", subcore_axis_name="s")

  @functools.partial(
      pl.kernel,
      out_type=jax.ShapeDtypeStruct((_NC, n, d), jnp.float32),
      mesh=mesh,
      scratch_types=[
          pltpu.VMEM((k_ch,), jnp.int32),
          pltpu.VMEM((k_ch,), jnp.int32),
          pltpu.VMEM((k_ch, d), jnp.float32),
          pltpu.VMEM((zr, d), jnp.float32),
          pltpu.VMEM_SHARED((n, d), jnp.float32),
      ],
      compiler_params=_sc_params(),
  )
  def k(y_hbm, src_hbm, dst_hbm, out_hbm, src_v, dst_v, rows_v, zero_v, acc_sh):
    c = lax.axis_index("c")
    s = lax.axis_index("s")

    @pl.loop(0, zr)
    def _(i):
      for j in range(d // _LANES):
        zero_v[i, pl.ds(j * _LANES, _LANES)] = jnp.zeros((_LANES,), jnp.float32)

    for t in range(nz):
      pltpu.sync_copy(zero_v, acc_sh.at[pl.ds(s * rpt + t * zr, zr)])

    @pl.when(s == 0)
    def _():
      pltpu.sync_copy(zero_v.at[pl.ds(0, rem)], acc_sh.at[pl.ds(_NS * rpt, rem)])

    plsc.subcore_barrier()

    base0 = (c * _NS + s) * per

    @pl.loop(0, nch)
    def _(i):
      base = base0 + i * k_ch
      pltpu.sync_copy(src_hbm.at[pl.ds(base, k_ch)], src_v)
      pltpu.sync_copy(dst_hbm.at[pl.ds(base, k_ch)], dst_v)
      pltpu.sync_copy(y_hbm.at[src_v], rows_v)
      pltpu.sync_copy(rows_v, acc_sh.at[dst_v], add=True)

    plsc.subcore_barrier()
    pltpu.sync_copy(
        acc_sh.at[pl.ds(s * rpt, rpt)],
        out_hbm.at[c, pl.ds(s * rpt, rpt)],
    )

    @pl.when(s == 0)
    def _():
      pltpu.sync_copy(
          acc_sh.at[pl.ds(_NS * rpt, rem)],
          out_hbm.at[c, pl.ds(_NS * rpt, rem)],
      )

  return k(y, src, dst)


_BLK = 1000


def _tc_dis(deg_t):
  n_pad, nw = deg_t.shape

  def body(deg_ref, dis_ref):
    deg = jnp.sum(deg_ref[...], axis=1, keepdims=True) + 1.0
    dis_ref[...] = lax.rsqrt(deg)

  return pl.pallas_call(
      body,
      out_shape=jax.ShapeDtypeStruct((n_pad, 1), jnp.float32),
  )(deg_t)


def _tc_matmul_scale(h, w, dis):
  n, d_in = h.shape
  d_out = w.shape[1]

  def body(h_ref, w_ref, d_ref, y_ref):
    mm = jnp.dot(h_ref[...], w_ref[...], preferred_element_type=jnp.float32)
    y_ref[...] = mm * d_ref[...]

  return pl.pallas_call(
      body,
      grid=(n // _BLK,),
      in_specs=[
          pl.BlockSpec((_BLK, d_in), lambda i: (i, 0)),
          pl.BlockSpec((d_in, d_out), lambda i: (0, 0)),
          pl.BlockSpec((_BLK, 1), lambda i: (i, 0)),
      ],
      out_specs=pl.BlockSpec((_BLK, d_out), lambda i: (i, 0)),
      out_shape=jax.ShapeDtypeStruct((n, d_out), jnp.float32),
  )(h, w, dis)


def _tc_layer(parts, y, dis, b, w_next):
  n, d = y.shape

  def body(p_ref, y_ref, d_ref, b_ref, w_ref, o_ref):
    t = (p_ref[0] + p_ref[1] + y_ref[...]) * d_ref[...] + b_ref[...]
    z = jnp.maximum(t, 0.0)
    o_ref[...] = (
        jnp.dot(z, w_ref[...], preferred_element_type=jnp.float32) * d_ref[...]
    )

  return pl.pallas_call(
      body,
      grid=(n // _BLK,),
      in_specs=[
          pl.BlockSpec((_NC, _BLK, d), lambda i: (0, i, 0)),
          pl.BlockSpec((_BLK, d), lambda i: (i, 0)),
          pl.BlockSpec((_BLK, 1), lambda i: (i, 0)),
          pl.BlockSpec((1, d), lambda i: (0, 0)),
          pl.BlockSpec((d, d), lambda i: (0, 0)),
      ],
      out_specs=pl.BlockSpec((_BLK, d), lambda i: (i, 0)),
      out_shape=jax.ShapeDtypeStruct((n, d), jnp.float32),
  )(parts, y, dis, b, w_next)


def _tc_final(parts, y, dis, b, batch_row, batch_col, gstats,
              a_mean, a_max, a_sum, a_st, mb1, m2, mb2, num_graphs):
  n, d = y.shape
  g = num_graphs
  mh = m2.shape[0]
  blk = 400
  nb = n // blk
  assert nb * blk == n

  def body(p_ref, y_ref, d_ref, b_ref, br_ref, bc_ref, gs_ref,
           am_ref, ax_ref, as_ref, ast_ref, mb1_ref, m2_ref, mb2_ref, o_ref,
           ssum_sc, smax_sc, cnt_sc):
    i = pl.program_id(0)

    @pl.when(i == 0)
    def _():
      ssum_sc[...] = jnp.zeros_like(ssum_sc)
      smax_sc[...] = jnp.full_like(smax_sc, -jnp.inf)
      cnt_sc[...] = jnp.zeros_like(cnt_sc)

    t = (p_ref[0] + p_ref[1] + y_ref[...]) * d_ref[...] + b_ref[...]
    h = jnp.maximum(t, 0.0)

    gid = lax.broadcasted_iota(jnp.int32, (g, 1), 0)
    oh = (br_ref[0] == gid).astype(jnp.float32)
    ssum_sc[...] += jnp.dot(oh, h, preferred_element_type=jnp.float32)
    cnt_sc[...] += jnp.sum(oh, axis=1, keepdims=True)

    bc = bc_ref[...]
    neg = jnp.float32(-jnp.inf)
    mx_rows = []
    for gg in range(g):
      m = jnp.where(bc == gg, h, neg)
      mx_rows.append(jnp.max(m, axis=0, keepdims=True))
    smax_sc[...] = jnp.maximum(smax_sc[...], jnp.concatenate(mx_rows, axis=0))

    @pl.when(i == nb - 1)
    def _():
      ssum = ssum_sc[...]
      mean = ssum / jnp.maximum(cnt_sc[...], 1.0)
      zpre = (
          jnp.dot(mean, am_ref[...], preferred_element_type=jnp.float32)
          + jnp.dot(smax_sc[...], ax_ref[...],
                    preferred_element_type=jnp.float32)
          + jnp.dot(ssum, as_ref[...], preferred_element_type=jnp.float32)
          + mb1_ref[...]
      )
      gs = gs_ref[...]
      for kk in range(gs.shape[1]):
        zpre = zpre + gs[:, kk:kk + 1] * ast_ref[kk:kk + 1, :]
      z = jnp.maximum(zpre, 0.0)
      o_ref[...] = (
          jnp.dot(z, m2_ref[...], preferred_element_type=jnp.float32)
          + mb2_ref[...]
      )

  return pl.pallas_call(
      body,
      grid=(nb,),
      in_specs=[
          pl.BlockSpec((_NC, blk, d), lambda i: (0, i, 0)),
          pl.BlockSpec((blk, d), lambda i: (i, 0)),
          pl.BlockSpec((blk, 1), lambda i: (i, 0)),
          pl.BlockSpec((1, d), lambda i: (0, 0)),
          pl.BlockSpec((1, 1, blk), lambda i: (i, 0, 0)),
          pl.BlockSpec((blk, 1), lambda i: (i, 0)),
          pl.BlockSpec((g, 3), lambda i: (0, 0)),
          pl.BlockSpec((d, mh), lambda i: (0, 0)),
          pl.BlockSpec((d, mh), lambda i: (0, 0)),
          pl.BlockSpec((d, mh), lambda i: (0, 0)),
          pl.BlockSpec((3, mh), lambda i: (0, 0)),
          pl.BlockSpec((1, mh), lambda i: (0, 0)),
          pl.BlockSpec((mh, 1), lambda i: (0, 0)),
          pl.BlockSpec((1, 1), lambda i: (0, 0)),
      ],
      out_specs=pl.BlockSpec((g, 1), lambda i: (0, 0)),
      out_shape=jax.ShapeDtypeStruct((g, 1), jnp.float32),
      scratch_shapes=[
          pltpu.VMEM((g, d), jnp.float32),
          pltpu.VMEM((g, d), jnp.float32),
          pltpu.VMEM((g, 1), jnp.float32),
      ],
  )(parts, y, dis, b, batch_row, batch_col, gstats,
    a_mean, a_max, a_sum, a_st, mb1, m2, mb2)


def kernel(x, edge_index, batch, graph_stats, W1, b1, W2, b2, W3, b3,
           M1, mb1, M2, mb2):
  n, d_in = x.shape
  h = W1.shape[1]
  g = graph_stats.shape[0]
  src = edge_index[0].astype(jnp.int32)
  dst = edge_index[1].astype(jnp.int32)

  n_pad = ((n + 16 * _LANES - 1) // (16 * _LANES)) * (16 * _LANES)
  deg_parts = _sc_degree(dst, n_pad)
  deg_t = deg_parts.reshape(_NC * _NS, n_pad).T

  dis_full = _tc_dis(deg_t)
  dis = dis_full[:n]
  y = _tc_matmul_scale(x, W1, dis)

  parts = _sc_scatter(y, src, dst)
  y = _tc_layer(parts, y, dis, b1.reshape(1, h), W2)
  parts = _sc_scatter(y, src, dst)
  y = _tc_layer(parts, y, dis, b2.reshape(1, h), W3)
  parts = _sc_scatter(y, src, dst)

  a_mean = M1[:h]
  a_max = M1[h:2 * h]
  a_sum = M1[2 * h:3 * h]
  a_st = M1[3 * h:]
  out = _tc_final(
      parts, y, dis, b3.reshape(1, h),
      batch.astype(jnp.int32).reshape(-1, 1, 400),
      batch.astype(jnp.int32).reshape(n, 1),
      graph_stats,
      a_mean, a_max, a_sum, a_st,
      mb1.reshape(1, -1), M2, mb2.reshape(1, 1), g,
  )
  return jnp.squeeze(out)


# speedup vs baseline: 22.2236x; 1.8791x over previous
import dataclasses
import functools

import jax
import jax.numpy as jnp
from jax import lax
from jax.experimental import pallas as pl
from jax.experimental.pallas import tpu as pltpu
from jax.experimental.pallas import tpu_sc as plsc

_NC = 2
_NS = 16
_LANES = 16


def _sc_params():
  cp = pltpu.CompilerParams()
  if "needs_layout_passes" in pltpu.CompilerParams.__dataclass_fields__:
    cp = dataclasses.replace(cp, needs_layout_passes=False)
  return cp


def _sc_degree(dst, n_pad):
  e = dst.shape[0]
  nw = _NC * _NS
  per = e // nw
  assert per * nw == e and per % _LANES == 0
  mesh = plsc.VectorSubcoreMesh(core_axis_name="c", subcore_axis_name="s")

  @functools.partial(
      pl.kernel,
      out_type=jax.ShapeDtypeStruct((nw * n_pad,), jnp.float32),
      mesh=mesh,
      scratch_types=[
          pltpu.VMEM((n_pad,), jnp.float32),
          pltpu.VMEM((per,), jnp.int32),
      ],
      compiler_params=_sc_params(),
  )
  def k(dst_hbm, out_hbm, hist_v, idx_v):
    c = lax.axis_index("c")
    s = lax.axis_index("s")
    w = c * _NS + s

    @pl.loop(0, n_pad // _LANES)
    def _(i):
      hist_v[pl.ds(i * _LANES, _LANES)] = jnp.zeros((_LANES,), jnp.float32)

    pltpu.sync_copy(dst_hbm.at[pl.ds(w * per, per)], idx_v)
    ones = jnp.ones((_LANES,), jnp.float32)

    @pl.loop(0, per // _LANES)
    def _(i):
      idx = idx_v[pl.ds(i * _LANES, _LANES)]
      plsc.addupdate_scatter(hist_v, [idx], ones)

    pltpu.sync_copy(hist_v, out_hbm.at[pl.ds(w * n_pad, n_pad)])

  return k(dst)


def _sc_scatter(y, src, dst):
  n, d = y.shape
  e = src.shape[0]
  nw = _NC * _NS
  per = e // nw
  k_ch = 80
  nch = per // k_ch
  assert per * nw == e and nch * k_ch == per and nch % 2 == 1
  rpt = (n // _NS) // 8 * 8
  rem = n - rpt * _NS
  assert rem % 8 == 0
  zr = 48
  nz = rpt // zr
  assert zr * nz == rpt and rem <= zr
  mesh = plsc.VectorSubcoreMesh(core_axis_name="c", subcore_axis_name="s")

  @functools.partial(
      pl.kernel,
      out_type=jax.ShapeDtypeStruct((_NC, n, d), jnp.float32),
      mesh=mesh,
      scratch_types=[
          pltpu.VMEM((per,), jnp.int32),
          pltpu.VMEM((k_ch,), jnp.int32),
          pltpu.VMEM((k_ch,), jnp.int32),
          pltpu.VMEM((k_ch, d), jnp.float32),
          pltpu.VMEM((k_ch, d), jnp.float32),
          pltpu.VMEM((zr, d), jnp.float32),
          pltpu.VMEM_SHARED((n, d), jnp.float32),
          pltpu.SemaphoreType.DMA,
          pltpu.SemaphoreType.DMA,
      ],
      compiler_params=_sc_params(),
  )
  def k(y_hbm, src_hbm, dst_hbm, out_hbm, src_v, dst_a, dst_b, buf_a, buf_b,
        zero_v, acc_sh, sem_a, sem_b):
    c = lax.axis_index("c")
    s = lax.axis_index("s")
    w = c * _NS + s

    @pl.loop(0, zr)
    def _(i):
      for j in range(d // _LANES):
        zero_v[i, pl.ds(j * _LANES, _LANES)] = jnp.zeros((_LANES,), jnp.float32)

    for t in range(nz):
      pltpu.sync_copy(zero_v, acc_sh.at[pl.ds(s * rpt + t * zr, zr)])

    @pl.when(s == 0)
    def _():
      pltpu.sync_copy(zero_v.at[pl.ds(0, rem)], acc_sh.at[pl.ds(_NS * rpt, rem)])

    base0 = w * per
    pltpu.sync_copy(src_hbm.at[pl.ds(base0, per)], src_v)
    plsc.subcore_barrier()

    def g_start(i, buf, sem):
      pltpu.async_copy(y_hbm.at[src_v.at[pl.ds(i * k_ch, k_ch)]], buf, sem)

    def g_wait(buf, sem):
      pltpu.make_async_copy(y_hbm.at[src_v.at[pl.ds(0, k_ch)]], buf, sem).wait()

    def d_load(i, dbuf):
      pltpu.sync_copy(dst_hbm.at[pl.ds(base0 + i * k_ch, k_ch)], dbuf)

    def s_add(buf, dbuf):
      pltpu.sync_copy(buf, acc_sh.at[dbuf], add=True)

    d_load(0, dst_a)
    g_start(0, buf_a, sem_a)

    @pl.loop(0, (nch - 1) // 2)
    def _(j):
      i = 2 * j
      d_load(i + 1, dst_b)
      g_wait(buf_a, sem_a)
      g_start(i + 1, buf_b, sem_b)
      s_add(buf_a, dst_a)
      d_load(i + 2, dst_a)
      g_start(i + 2, buf_a, sem_a)
      g_wait(buf_b, sem_b)
      s_add(buf_b, dst_b)

    g_wait(buf_a, sem_a)
    s_add(buf_a, dst_a)

    plsc.subcore_barrier()
    pltpu.sync_copy(
        acc_sh.at[pl.ds(s * rpt, rpt)],
        out_hbm.at[c, pl.ds(s * rpt, rpt)],
    )

    @pl.when(s == 0)
    def _():
      pltpu.sync_copy(
          acc_sh.at[pl.ds(_NS * rpt, rem)],
          out_hbm.at[c, pl.ds(_NS * rpt, rem)],
      )

  return k(y, src, dst)


_BLK = 1000


def _tc_dis(deg_t):
  n_pad, nw = deg_t.shape

  def body(deg_ref, dis_ref):
    deg = jnp.sum(deg_ref[...], axis=1, keepdims=True) + 1.0
    dis_ref[...] = lax.rsqrt(deg)

  return pl.pallas_call(
      body,
      out_shape=jax.ShapeDtypeStruct((n_pad, 1), jnp.float32),
  )(deg_t)


def _tc_matmul_scale(h, w, dis):
  n, d_in = h.shape
  d_out = w.shape[1]

  def body(h_ref, w_ref, d_ref, y_ref):
    mm = jnp.dot(h_ref[...], w_ref[...], preferred_element_type=jnp.float32)
    y_ref[...] = mm * d_ref[...]

  return pl.pallas_call(
      body,
      grid=(n // _BLK,),
      in_specs=[
          pl.BlockSpec((_BLK, d_in), lambda i: (i, 0)),
          pl.BlockSpec((d_in, d_out), lambda i: (0, 0)),
          pl.BlockSpec((_BLK, 1), lambda i: (i, 0)),
      ],
      out_specs=pl.BlockSpec((_BLK, d_out), lambda i: (i, 0)),
      out_shape=jax.ShapeDtypeStruct((n, d_out), jnp.float32),
  )(h, w, dis)


def _tc_layer(parts, y, dis, b, w_next):
  n, d = y.shape

  def body(p_ref, y_ref, d_ref, b_ref, w_ref, o_ref):
    t = (p_ref[0] + p_ref[1] + y_ref[...]) * d_ref[...] + b_ref[...]
    z = jnp.maximum(t, 0.0)
    o_ref[...] = (
        jnp.dot(z, w_ref[...], preferred_element_type=jnp.float32) * d_ref[...]
    )

  return pl.pallas_call(
      body,
      grid=(n // _BLK,),
      in_specs=[
          pl.BlockSpec((_NC, _BLK, d), lambda i: (0, i, 0)),
          pl.BlockSpec((_BLK, d), lambda i: (i, 0)),
          pl.BlockSpec((_BLK, 1), lambda i: (i, 0)),
          pl.BlockSpec((1, d), lambda i: (0, 0)),
          pl.BlockSpec((d, d), lambda i: (0, 0)),
      ],
      out_specs=pl.BlockSpec((_BLK, d), lambda i: (i, 0)),
      out_shape=jax.ShapeDtypeStruct((n, d), jnp.float32),
  )(parts, y, dis, b, w_next)


def _tc_final(parts, y, dis, b, batch_row, batch_col, gstats,
              a_mean, a_max, a_sum, a_st, mb1, m2, mb2, num_graphs):
  n, d = y.shape
  g = num_graphs
  mh = m2.shape[0]
  blk = 400
  nb = n // blk
  assert nb * blk == n

  def body(p_ref, y_ref, d_ref, b_ref, br_ref, bc_ref, gs_ref,
           am_ref, ax_ref, as_ref, ast_ref, mb1_ref, m2_ref, mb2_ref, o_ref,
           ssum_sc, smax_sc, cnt_sc):
    i = pl.program_id(0)

    @pl.when(i == 0)
    def _():
      ssum_sc[...] = jnp.zeros_like(ssum_sc)
      smax_sc[...] = jnp.full_like(smax_sc, -jnp.inf)
      cnt_sc[...] = jnp.zeros_like(cnt_sc)

    t = (p_ref[0] + p_ref[1] + y_ref[...]) * d_ref[...] + b_ref[...]
    h = jnp.maximum(t, 0.0)

    gid = lax.broadcasted_iota(jnp.int32, (g, 1), 0)
    oh = (br_ref[0] == gid).astype(jnp.float32)
    ssum_sc[...] += jnp.dot(oh, h, preferred_element_type=jnp.float32)
    cnt_sc[...] += jnp.sum(oh, axis=1, keepdims=True)

    bc = bc_ref[...]
    neg = jnp.float32(-jnp.inf)
    mx_rows = []
    for gg in range(g):
      m = jnp.where(bc == gg, h, neg)
      mx_rows.append(jnp.max(m, axis=0, keepdims=True))
    smax_sc[...] = jnp.maximum(smax_sc[...], jnp.concatenate(mx_rows, axis=0))

    @pl.when(i == nb - 1)
    def _():
      ssum = ssum_sc[...]
      mean = ssum / jnp.maximum(cnt_sc[...], 1.0)
      zpre = (
          jnp.dot(mean, am_ref[...], preferred_element_type=jnp.float32)
          + jnp.dot(smax_sc[...], ax_ref[...],
                    preferred_element_type=jnp.float32)
          + jnp.dot(ssum, as_ref[...], preferred_element_type=jnp.float32)
          + mb1_ref[...]
      )
      gs = gs_ref[...]
      for kk in range(gs.shape[1]):
        zpre = zpre + gs[:, kk:kk + 1] * ast_ref[kk:kk + 1, :]
      z = jnp.maximum(zpre, 0.0)
      o_ref[...] = (
          jnp.dot(z, m2_ref[...], preferred_element_type=jnp.float32)
          + mb2_ref[...]
      )

  return pl.pallas_call(
      body,
      grid=(nb,),
      in_specs=[
          pl.BlockSpec((_NC, blk, d), lambda i: (0, i, 0)),
          pl.BlockSpec((blk, d), lambda i: (i, 0)),
          pl.BlockSpec((blk, 1), lambda i: (i, 0)),
          pl.BlockSpec((1, d), lambda i: (0, 0)),
          pl.BlockSpec((1, 1, blk), lambda i: (i, 0, 0)),
          pl.BlockSpec((blk, 1), lambda i: (i, 0)),
          pl.BlockSpec((g, 3), lambda i: (0, 0)),
          pl.BlockSpec((d, mh), lambda i: (0, 0)),
          pl.BlockSpec((d, mh), lambda i: (0, 0)),
          pl.BlockSpec((d, mh), lambda i: (0, 0)),
          pl.BlockSpec((3, mh), lambda i: (0, 0)),
          pl.BlockSpec((1, mh), lambda i: (0, 0)),
          pl.BlockSpec((mh, 1), lambda i: (0, 0)),
          pl.BlockSpec((1, 1), lambda i: (0, 0)),
      ],
      out_specs=pl.BlockSpec((g, 1), lambda i: (0, 0)),
      out_shape=jax.ShapeDtypeStruct((g, 1), jnp.float32),
      scratch_shapes=[
          pltpu.VMEM((g, d), jnp.float32),
          pltpu.VMEM((g, d), jnp.float32),
          pltpu.VMEM((g, 1), jnp.float32),
      ],
  )(parts, y, dis, b, batch_row, batch_col, gstats,
    a_mean, a_max, a_sum, a_st, mb1, m2, mb2)


def kernel(x, edge_index, batch, graph_stats, W1, b1, W2, b2, W3, b3,
           M1, mb1, M2, mb2):
  n, d_in = x.shape
  h = W1.shape[1]
  g = graph_stats.shape[0]
  src = edge_index[0].astype(jnp.int32)
  dst = edge_index[1].astype(jnp.int32)

  n_pad = ((n + 16 * _LANES - 1) // (16 * _LANES)) * (16 * _LANES)
  deg_parts = _sc_degree(dst, n_pad)
  deg_t = deg_parts.reshape(_NC * _NS, n_pad).T

  dis_full = _tc_dis(deg_t)
  dis = dis_full[:n]
  y = _tc_matmul_scale(x, W1, dis)

  parts = _sc_scatter(y, src, dst)
  y = _tc_layer(parts, y, dis, b1.reshape(1, h), W2)
  parts = _sc_scatter(y, src, dst)
  y = _tc_layer(parts, y, dis, b2.reshape(1, h), W3)
  parts = _sc_scatter(y, src, dst)

  a_mean = M1[:h]
  a_max = M1[h:2 * h]
  a_sum = M1[2 * h:3 * h]
  a_st = M1[3 * h:]
  out = _tc_final(
      parts, y, dis, b3.reshape(1, h),
      batch.astype(jnp.int32).reshape(-1, 1, 400),
      batch.astype(jnp.int32).reshape(n, 1),
      graph_stats,
      a_mean, a_max, a_sum, a_st,
      mb1.reshape(1, -1), M2, mb2.reshape(1, 1), g,
  )
  return jnp.squeeze(out)


# speedup vs baseline: 25.6546x; 1.1544x over previous
import dataclasses
import functools

import jax
import jax.numpy as jnp
from jax import lax
from jax.experimental import pallas as pl
from jax.experimental.pallas import tpu as pltpu
from jax.experimental.pallas import tpu_sc as plsc

_NC = 2
_NS = 16
_LANES = 16


def _sc_params():
  cp = pltpu.CompilerParams()
  if "needs_layout_passes" in pltpu.CompilerParams.__dataclass_fields__:
    cp = dataclasses.replace(cp, needs_layout_passes=False)
  return cp


def _sc_degree(dst, n_pad):
  e = dst.shape[0]
  nw = _NC * _NS
  per = e // nw
  assert per * nw == e and per % _LANES == 0
  mesh = plsc.VectorSubcoreMesh(core_axis_name="c", subcore_axis_name="s")

  @functools.partial(
      pl.kernel,
      out_type=jax.ShapeDtypeStruct((nw * n_pad,), jnp.float32),
      mesh=mesh,
      scratch_types=[
          pltpu.VMEM((n_pad,), jnp.float32),
          pltpu.VMEM((per,), jnp.int32),
      ],
      compiler_params=_sc_params(),
  )
  def k(dst_hbm, out_hbm, hist_v, idx_v):
    c = lax.axis_index("c")
    s = lax.axis_index("s")
    w = c * _NS + s

    @pl.loop(0, n_pad // _LANES)
    def _(i):
      hist_v[pl.ds(i * _LANES, _LANES)] = jnp.zeros((_LANES,), jnp.float32)

    pltpu.sync_copy(dst_hbm.at[pl.ds(w * per, per)], idx_v)
    ones = jnp.ones((_LANES,), jnp.float32)

    @pl.loop(0, per // _LANES)
    def _(i):
      idx = idx_v[pl.ds(i * _LANES, _LANES)]
      plsc.addupdate_scatter(hist_v, [idx], ones)

    pltpu.sync_copy(hist_v, out_hbm.at[pl.ds(w * n_pad, n_pad)])

  return k(dst)


def _sc_scatter(y, src, dst):
  n, d = y.shape
  e = src.shape[0]
  nw = _NC * _NS
  per = e // nw
  k_ch = 80
  nch = per // k_ch
  assert per * nw == e and nch * k_ch == per and (nch - 5) % 3 == 0
  rpt = (n // _NS) // 8 * 8
  rem = n - rpt * _NS
  assert rem % 8 == 0
  zr = 48
  nz = rpt // zr
  assert zr * nz == rpt and rem <= zr
  mesh = plsc.VectorSubcoreMesh(core_axis_name="c", subcore_axis_name="s")

  @functools.partial(
      pl.kernel,
      out_type=jax.ShapeDtypeStruct((_NC, n, d), jnp.float32),
      mesh=mesh,
      scratch_types=[
          pltpu.VMEM((per,), jnp.int32),
          pltpu.VMEM((k_ch,), jnp.int32),
          pltpu.VMEM((k_ch,), jnp.int32),
          pltpu.VMEM((k_ch,), jnp.int32),
          pltpu.VMEM((k_ch, d), jnp.float32),
          pltpu.VMEM((k_ch, d), jnp.float32),
          pltpu.VMEM((k_ch, d), jnp.float32),
          pltpu.VMEM((zr, d), jnp.float32),
          pltpu.VMEM_SHARED((n, d), jnp.float32),
          pltpu.SemaphoreType.DMA,
          pltpu.SemaphoreType.DMA,
          pltpu.SemaphoreType.DMA,
          pltpu.SemaphoreType.DMA,
          pltpu.SemaphoreType.DMA,
          pltpu.SemaphoreType.DMA,
      ],
      compiler_params=_sc_params(),
  )
  def k(y_hbm, src_hbm, dst_hbm, out_hbm, src_v, dst_a, dst_b, dst_c,
        buf_a, buf_b, buf_c, zero_v, acc_sh,
        gsem_a, gsem_b, gsem_c, ssem_a, ssem_b, ssem_c):
    c = lax.axis_index("c")
    s = lax.axis_index("s")
    w = c * _NS + s
    bufs = (buf_a, buf_b, buf_c)
    dsts = (dst_a, dst_b, dst_c)
    gsems = (gsem_a, gsem_b, gsem_c)
    ssems = (ssem_a, ssem_b, ssem_c)

    @pl.loop(0, zr)
    def _(i):
      for j in range(d // _LANES):
        zero_v[i, pl.ds(j * _LANES, _LANES)] = jnp.zeros((_LANES,), jnp.float32)

    for t in range(nz):
      pltpu.sync_copy(zero_v, acc_sh.at[pl.ds(s * rpt + t * zr, zr)])

    @pl.when(s == 0)
    def _():
      pltpu.sync_copy(zero_v.at[pl.ds(0, rem)], acc_sh.at[pl.ds(_NS * rpt, rem)])

    base0 = w * per
    pltpu.sync_copy(src_hbm.at[pl.ds(base0, per)], src_v)
    plsc.subcore_barrier()

    def g_start(i, p):
      pltpu.async_copy(y_hbm.at[src_v.at[pl.ds(i * k_ch, k_ch)]],
                       bufs[p], gsems[p])

    def g_wait(p):
      pltpu.make_async_copy(y_hbm.at[src_v.at[pl.ds(0, k_ch)]],
                            bufs[p], gsems[p]).wait()

    def d_load(i, p):
      pltpu.sync_copy(dst_hbm.at[pl.ds(base0 + i * k_ch, k_ch)], dsts[p])

    def s_start(p):
      pltpu.async_copy(bufs[p], acc_sh.at[dsts[p]], ssems[p], add=True)

    def s_wait(p):
      pltpu.make_async_copy(bufs[p], acc_sh.at[dsts[p]], ssems[p]).wait()

    d_load(0, 0)
    g_start(0, 0)
    d_load(1, 1)
    g_start(1, 1)

    def step(i, p, first_round):
      g_wait(p)
      s_start(p)
      q = (p + 2) % 3
      if not (first_round and q == 2):
        s_wait(q)
      d_load(i + 2, q)
      g_start(i + 2, q)

    step(0, 0, True)
    step(1, 1, True)
    step(2, 2, False)

    @pl.loop(0, (nch - 5) // 3)
    def _(j):
      base = 3 + 3 * j
      step(base + 0, 0, False)
      step(base + 1, 1, False)
      step(base + 2, 2, False)

    g_wait(0)
    s_start(0)
    g_wait(1)
    s_start(1)
    s_wait(0)
    s_wait(1)
    s_wait(2)

    plsc.subcore_barrier()
    pltpu.sync_copy(
        acc_sh.at[pl.ds(s * rpt, rpt)],
        out_hbm.at[c, pl.ds(s * rpt, rpt)],
    )

    @pl.when(s == 0)
    def _():
      pltpu.sync_copy(
          acc_sh.at[pl.ds(_NS * rpt, rem)],
          out_hbm.at[c, pl.ds(_NS * rpt, rem)],
      )

  return k(y, src, dst)


_BLK = 1000


def _tc_dis(deg_t):
  n_pad, nw = deg_t.shape

  def body(deg_ref, dis_ref):
    deg = jnp.sum(deg_ref[...], axis=1, keepdims=True) + 1.0
    dis_ref[...] = lax.rsqrt(deg)

  return pl.pallas_call(
      body,
      out_shape=jax.ShapeDtypeStruct((n_pad, 1), jnp.float32),
  )(deg_t)


def _tc_matmul_scale(h, w, dis):
  n, d_in = h.shape
  d_out = w.shape[1]

  def body(h_ref, w_ref, d_ref, y_ref):
    mm = jnp.dot(h_ref[...], w_ref[...], preferred_element_type=jnp.float32)
    y_ref[...] = mm * d_ref[...]

  return pl.pallas_call(
      body,
      grid=(n // _BLK,),
      in_specs=[
          pl.BlockSpec((_BLK, d_in), lambda i: (i, 0)),
          pl.BlockSpec((d_in, d_out), lambda i: (0, 0)),
          pl.BlockSpec((_BLK, 1), lambda i: (i, 0)),
      ],
      out_specs=pl.BlockSpec((_BLK, d_out), lambda i: (i, 0)),
      out_shape=jax.ShapeDtypeStruct((n, d_out), jnp.float32),
  )(h, w, dis)


def _tc_layer(parts, y, dis, b, w_next):
  n, d = y.shape

  def body(p_ref, y_ref, d_ref, b_ref, w_ref, o_ref):
    t = (p_ref[0] + p_ref[1] + y_ref[...]) * d_ref[...] + b_ref[...]
    z = jnp.maximum(t, 0.0)
    o_ref[...] = (
        jnp.dot(z, w_ref[...], preferred_element_type=jnp.float32) * d_ref[...]
    )

  return pl.pallas_call(
      body,
      grid=(n // _BLK,),
      in_specs=[
          pl.BlockSpec((_NC, _BLK, d), lambda i: (0, i, 0)),
          pl.BlockSpec((_BLK, d), lambda i: (i, 0)),
          pl.BlockSpec((_BLK, 1), lambda i: (i, 0)),
          pl.BlockSpec((1, d), lambda i: (0, 0)),
          pl.BlockSpec((d, d), lambda i: (0, 0)),
      ],
      out_specs=pl.BlockSpec((_BLK, d), lambda i: (i, 0)),
      out_shape=jax.ShapeDtypeStruct((n, d), jnp.float32),
  )(parts, y, dis, b, w_next)


def _tc_final(parts, y, dis, b, batch_row, batch_col, gstats,
              a_mean, a_max, a_sum, a_st, mb1, m2, mb2, num_graphs):
  n, d = y.shape
  g = num_graphs
  mh = m2.shape[0]
  blk = 400
  nb = n // blk
  assert nb * blk == n

  def body(p_ref, y_ref, d_ref, b_ref, br_ref, bc_ref, gs_ref,
           am_ref, ax_ref, as_ref, ast_ref, mb1_ref, m2_ref, mb2_ref, o_ref,
           ssum_sc, smax_sc, cnt_sc):
    i = pl.program_id(0)

    @pl.when(i == 0)
    def _():
      ssum_sc[...] = jnp.zeros_like(ssum_sc)
      smax_sc[...] = jnp.full_like(smax_sc, -jnp.inf)
      cnt_sc[...] = jnp.zeros_like(cnt_sc)

    t = (p_ref[0] + p_ref[1] + y_ref[...]) * d_ref[...] + b_ref[...]
    h = jnp.maximum(t, 0.0)

    gid = lax.broadcasted_iota(jnp.int32, (g, 1), 0)
    oh = (br_ref[0] == gid).astype(jnp.float32)
    ssum_sc[...] += jnp.dot(oh, h, preferred_element_type=jnp.float32)
    cnt_sc[...] += jnp.sum(oh, axis=1, keepdims=True)

    bc = bc_ref[...]
    neg = jnp.float32(-jnp.inf)
    mx_rows = []
    for gg in range(g):
      m = jnp.where(bc == gg, h, neg)
      mx_rows.append(jnp.max(m, axis=0, keepdims=True))
    smax_sc[...] = jnp.maximum(smax_sc[...], jnp.concatenate(mx_rows, axis=0))

    @pl.when(i == nb - 1)
    def _():
      ssum = ssum_sc[...]
      mean = ssum / jnp.maximum(cnt_sc[...], 1.0)
      zpre = (
          jnp.dot(mean, am_ref[...], preferred_element_type=jnp.float32)
          + jnp.dot(smax_sc[...], ax_ref[...],
                    preferred_element_type=jnp.float32)
          + jnp.dot(ssum, as_ref[...], preferred_element_type=jnp.float32)
          + mb1_ref[...]
      )
      gs = gs_ref[...]
      for kk in range(gs.shape[1]):
        zpre = zpre + gs[:, kk:kk + 1] * ast_ref[kk:kk + 1, :]
      z = jnp.maximum(zpre, 0.0)
      o_ref[...] = (
          jnp.dot(z, m2_ref[...], preferred_element_type=jnp.float32)
          + mb2_ref[...]
      )

  return pl.pallas_call(
      body,
      grid=(nb,),
      in_specs=[
          pl.BlockSpec((_NC, blk, d), lambda i: (0, i, 0)),
          pl.BlockSpec((blk, d), lambda i: (i, 0)),
          pl.BlockSpec((blk, 1), lambda i: (i, 0)),
          pl.BlockSpec((1, d), lambda i: (0, 0)),
          pl.BlockSpec((1, 1, blk), lambda i: (i, 0, 0)),
          pl.BlockSpec((blk, 1), lambda i: (i, 0)),
          pl.BlockSpec((g, 3), lambda i: (0, 0)),
          pl.BlockSpec((d, mh), lambda i: (0, 0)),
          pl.BlockSpec((d, mh), lambda i: (0, 0)),
          pl.BlockSpec((d, mh), lambda i: (0, 0)),
          pl.BlockSpec((3, mh), lambda i: (0, 0)),
          pl.BlockSpec((1, mh), lambda i: (0, 0)),
          pl.BlockSpec((mh, 1), lambda i: (0, 0)),
          pl.BlockSpec((1, 1), lambda i: (0, 0)),
      ],
      out_specs=pl.BlockSpec((g, 1), lambda i: (0, 0)),
      out_shape=jax.ShapeDtypeStruct((g, 1), jnp.float32),
      scratch_shapes=[
          pltpu.VMEM((g, d), jnp.float32),
          pltpu.VMEM((g, d), jnp.float32),
          pltpu.VMEM((g, 1), jnp.float32),
      ],
  )(parts, y, dis, b, batch_row, batch_col, gstats,
    a_mean, a_max, a_sum, a_st, mb1, m2, mb2)


def kernel(x, edge_index, batch, graph_stats, W1, b1, W2, b2, W3, b3,
           M1, mb1, M2, mb2):
  n, d_in = x.shape
  h = W1.shape[1]
  g = graph_stats.shape[0]
  src = edge_index[0].astype(jnp.int32)
  dst = edge_index[1].astype(jnp.int32)

  n_pad = ((n + 16 * _LANES - 1) // (16 * _LANES)) * (16 * _LANES)
  deg_parts = _sc_degree(dst, n_pad)
  deg_t = deg_parts.reshape(_NC * _NS, n_pad).T

  dis_full = _tc_dis(deg_t)
  dis = dis_full[:n]
  y = _tc_matmul_scale(x, W1, dis)

  parts = _sc_scatter(y, src, dst)
  y = _tc_layer(parts, y, dis, b1.reshape(1, h), W2)
  parts = _sc_scatter(y, src, dst)
  y = _tc_layer(parts, y, dis, b2.reshape(1, h), W3)
  parts = _sc_scatter(y, src, dst)

  a_mean = M1[:h]
  a_max = M1[h:2 * h]
  a_sum = M1[2 * h:3 * h]
  a_st = M1[3 * h:]
  out = _tc_final(
      parts, y, dis, b3.reshape(1, h),
      batch.astype(jnp.int32).reshape(-1, 1, 400),
      batch.astype(jnp.int32).reshape(n, 1),
      graph_stats,
      a_mean, a_max, a_sum, a_st,
      mb1.reshape(1, -1), M2, mb2.reshape(1, 1), g,
  )
  return jnp.squeeze(out)


# speedup vs baseline: 29.6114x; 1.1542x over previous
import dataclasses
import functools

import jax
import jax.numpy as jnp
from jax import lax
from jax.experimental import pallas as pl
from jax.experimental.pallas import tpu as pltpu
from jax.experimental.pallas import tpu_sc as plsc

_NC = 2
_NS = 16
_LANES = 16


def _sc_params():
  cp = pltpu.CompilerParams()
  if "needs_layout_passes" in pltpu.CompilerParams.__dataclass_fields__:
    cp = dataclasses.replace(cp, needs_layout_passes=False)
  return cp


def _sc_degree(dst, n_pad):
  e = dst.shape[0]
  nw = _NC * _NS
  per = e // nw
  assert per * nw == e and per % _LANES == 0
  mesh = plsc.VectorSubcoreMesh(core_axis_name="c", subcore_axis_name="s")

  @functools.partial(
      pl.kernel,
      out_type=jax.ShapeDtypeStruct((nw * n_pad,), jnp.float32),
      mesh=mesh,
      scratch_types=[
          pltpu.VMEM((n_pad,), jnp.float32),
          pltpu.VMEM((per,), jnp.int32),
      ],
      compiler_params=_sc_params(),
  )
  def k(dst_hbm, out_hbm, hist_v, idx_v):
    c = lax.axis_index("c")
    s = lax.axis_index("s")
    w = c * _NS + s

    @pl.loop(0, n_pad // _LANES)
    def _(i):
      hist_v[pl.ds(i * _LANES, _LANES)] = jnp.zeros((_LANES,), jnp.float32)

    pltpu.sync_copy(dst_hbm.at[pl.ds(w * per, per)], idx_v)
    ones = jnp.ones((_LANES,), jnp.float32)

    @pl.loop(0, per // _LANES)
    def _(i):
      idx = idx_v[pl.ds(i * _LANES, _LANES)]
      plsc.addupdate_scatter(hist_v, [idx], ones)

    pltpu.sync_copy(hist_v, out_hbm.at[pl.ds(w * n_pad, n_pad)])

  return k(dst)


def _sc_scatter(y, src, dst):
  n, d = y.shape
  e = src.shape[0]
  nw = _NC * _NS
  per = e // nw
  k_ch = 80
  nch = per // k_ch
  assert per * nw == e and nch * k_ch == per and (nch - 5) % 3 == 0
  rpt = (n // _NS) // 8 * 8
  rem = n - rpt * _NS
  assert rem % 8 == 0
  zr = 48
  nz = rpt // zr
  assert zr * nz == rpt and rem <= zr
  mesh = plsc.VectorSubcoreMesh(core_axis_name="c", subcore_axis_name="s")

  @functools.partial(
      pl.kernel,
      out_type=jax.ShapeDtypeStruct((_NC, n, d), jnp.float32),
      mesh=mesh,
      scratch_types=[
          pltpu.VMEM((per,), jnp.int32),
          pltpu.VMEM((k_ch,), jnp.int32),
          pltpu.VMEM((k_ch,), jnp.int32),
          pltpu.VMEM((k_ch,), jnp.int32),
          pltpu.VMEM((k_ch, d), jnp.float32),
          pltpu.VMEM((k_ch, d), jnp.float32),
          pltpu.VMEM((k_ch, d), jnp.float32),
          pltpu.VMEM((zr, d), jnp.float32),
          pltpu.VMEM_SHARED((n, d), jnp.float32),
          pltpu.SemaphoreType.DMA,
          pltpu.SemaphoreType.DMA,
          pltpu.SemaphoreType.DMA,
          pltpu.SemaphoreType.DMA,
          pltpu.SemaphoreType.DMA,
          pltpu.SemaphoreType.DMA,
          pltpu.SemaphoreType.DMA,
          pltpu.SemaphoreType.DMA,
          pltpu.SemaphoreType.DMA,
      ],
      compiler_params=_sc_params(),
  )
  def k(y_hbm, src_hbm, dst_hbm, out_hbm, src_v, dst_a, dst_b, dst_c,
        buf_a, buf_b, buf_c, zero_v, acc_sh,
        gsem_a, gsem_b, gsem_c, ssem_a, ssem_b, ssem_c,
        dsem_a, dsem_b, dsem_c):
    c = lax.axis_index("c")
    s = lax.axis_index("s")
    w = c * _NS + s
    bufs = (buf_a, buf_b, buf_c)
    dsts = (dst_a, dst_b, dst_c)
    gsems = (gsem_a, gsem_b, gsem_c)
    ssems = (ssem_a, ssem_b, ssem_c)
    dsems = (dsem_a, dsem_b, dsem_c)

    @pl.loop(0, zr)
    def _(i):
      for j in range(d // _LANES):
        zero_v[i, pl.ds(j * _LANES, _LANES)] = jnp.zeros((_LANES,), jnp.float32)

    for t in range(nz):
      pltpu.sync_copy(zero_v, acc_sh.at[pl.ds(s * rpt + t * zr, zr)])

    @pl.when(s == 0)
    def _():
      pltpu.sync_copy(zero_v.at[pl.ds(0, rem)], acc_sh.at[pl.ds(_NS * rpt, rem)])

    base0 = w * per
    pltpu.sync_copy(src_hbm.at[pl.ds(base0, per)], src_v)
    plsc.subcore_barrier()

    def g_start(i, p):
      pltpu.async_copy(y_hbm.at[src_v.at[pl.ds(i * k_ch, k_ch)]],
                       bufs[p], gsems[p])

    def g_wait(p):
      pltpu.make_async_copy(y_hbm.at[src_v.at[pl.ds(0, k_ch)]],
                            bufs[p], gsems[p]).wait()

    def d_start(i, p):
      pltpu.async_copy(dst_hbm.at[pl.ds(base0 + i * k_ch, k_ch)],
                       dsts[p], dsems[p])

    def d_wait(p):
      pltpu.make_async_copy(dst_hbm.at[pl.ds(base0, k_ch)],
                            dsts[p], dsems[p]).wait()

    def s_start(p):
      pltpu.async_copy(bufs[p], acc_sh.at[dsts[p]], ssems[p], add=True)

    def s_wait(p):
      pltpu.make_async_copy(bufs[p], acc_sh.at[dsts[p]], ssems[p]).wait()

    d_start(0, 0)
    g_start(0, 0)
    d_start(1, 1)
    g_start(1, 1)

    def step(i, p, first_round):
      g_wait(p)
      d_wait(p)
      s_start(p)
      q = (p + 2) % 3
      if not (first_round and q == 2):
        s_wait(q)
      d_start(i + 2, q)
      g_start(i + 2, q)

    step(0, 0, True)
    step(1, 1, True)
    step(2, 2, False)

    @pl.loop(0, (nch - 5) // 3)
    def _(j):
      base = 3 + 3 * j
      step(base + 0, 0, False)
      step(base + 1, 1, False)
      step(base + 2, 2, False)

    g_wait(0)
    d_wait(0)
    s_start(0)
    g_wait(1)
    d_wait(1)
    s_start(1)
    s_wait(0)
    s_wait(1)
    s_wait(2)

    plsc.subcore_barrier()
    pltpu.sync_copy(
        acc_sh.at[pl.ds(s * rpt, rpt)],
        out_hbm.at[c, pl.ds(s * rpt, rpt)],
    )

    @pl.when(s == 0)
    def _():
      pltpu.sync_copy(
          acc_sh.at[pl.ds(_NS * rpt, rem)],
          out_hbm.at[c, pl.ds(_NS * rpt, rem)],
      )

  return k(y, src, dst)


_BLK = 1000


def _tc_dis(deg_t):
  n_pad, nw = deg_t.shape

  def body(deg_ref, dis_ref):
    deg = jnp.sum(deg_ref[...], axis=1, keepdims=True) + 1.0
    dis_ref[...] = lax.rsqrt(deg)

  return pl.pallas_call(
      body,
      out_shape=jax.ShapeDtypeStruct((n_pad, 1), jnp.float32),
  )(deg_t)


def _tc_matmul_scale(h, w, dis):
  n, d_in = h.shape
  d_out = w.shape[1]

  def body(h_ref, w_ref, d_ref, y_ref):
    mm = jnp.dot(h_ref[...], w_ref[...], preferred_element_type=jnp.float32)
    y_ref[...] = mm * d_ref[...]

  return pl.pallas_call(
      body,
      grid=(n // _BLK,),
      in_specs=[
          pl.BlockSpec((_BLK, d_in), lambda i: (i, 0)),
          pl.BlockSpec((d_in, d_out), lambda i: (0, 0)),
          pl.BlockSpec((_BLK, 1), lambda i: (i, 0)),
      ],
      out_specs=pl.BlockSpec((_BLK, d_out), lambda i: (i, 0)),
      out_shape=jax.ShapeDtypeStruct((n, d_out), jnp.float32),
  )(h, w, dis)


def _tc_layer(parts, y, dis, b, w_next):
  n, d = y.shape

  def body(p_ref, y_ref, d_ref, b_ref, w_ref, o_ref):
    t = (p_ref[0] + p_ref[1] + y_ref[...]) * d_ref[...] + b_ref[...]
    z = jnp.maximum(t, 0.0)
    o_ref[...] = (
        jnp.dot(z, w_ref[...], preferred_element_type=jnp.float32) * d_ref[...]
    )

  return pl.pallas_call(
      body,
      grid=(n // _BLK,),
      in_specs=[
          pl.BlockSpec((_NC, _BLK, d), lambda i: (0, i, 0)),
          pl.BlockSpec((_BLK, d), lambda i: (i, 0)),
          pl.BlockSpec((_BLK, 1), lambda i: (i, 0)),
          pl.BlockSpec((1, d), lambda i: (0, 0)),
          pl.BlockSpec((d, d), lambda i: (0, 0)),
      ],
      out_specs=pl.BlockSpec((_BLK, d), lambda i: (i, 0)),
      out_shape=jax.ShapeDtypeStruct((n, d), jnp.float32),
  )(parts, y, dis, b, w_next)


def _tc_final(parts, y, dis, b, batch_row, batch_col, gstats,
              a_mean, a_max, a_sum, a_st, mb1, m2, mb2, num_graphs):
  n, d = y.shape
  g = num_graphs
  mh = m2.shape[0]
  blk = 400
  nb = n // blk
  assert nb * blk == n

  def body(p_ref, y_ref, d_ref, b_ref, br_ref, bc_ref, gs_ref,
           am_ref, ax_ref, as_ref, ast_ref, mb1_ref, m2_ref, mb2_ref, o_ref,
           ssum_sc, smax_sc, cnt_sc):
    i = pl.program_id(0)

    @pl.when(i == 0)
    def _():
      ssum_sc[...] = jnp.zeros_like(ssum_sc)
      smax_sc[...] = jnp.full_like(smax_sc, -jnp.inf)
      cnt_sc[...] = jnp.zeros_like(cnt_sc)

    t = (p_ref[0] + p_ref[1] + y_ref[...]) * d_ref[...] + b_ref[...]
    h = jnp.maximum(t, 0.0)

    gid = lax.broadcasted_iota(jnp.int32, (g, 1), 0)
    oh = (br_ref[0] == gid).astype(jnp.float32)
    ssum_sc[...] += jnp.dot(oh, h, preferred_element_type=jnp.float32)
    cnt_sc[...] += jnp.sum(oh, axis=1, keepdims=True)

    bc = bc_ref[...]
    neg = jnp.float32(-jnp.inf)
    mx_rows = []
    for gg in range(g):
      m = jnp.where(bc == gg, h, neg)
      mx_rows.append(jnp.max(m, axis=0, keepdims=True))
    smax_sc[...] = jnp.maximum(smax_sc[...], jnp.concatenate(mx_rows, axis=0))

    @pl.when(i == nb - 1)
    def _():
      ssum = ssum_sc[...]
      mean = ssum / jnp.maximum(cnt_sc[...], 1.0)
      zpre = (
          jnp.dot(mean, am_ref[...], preferred_element_type=jnp.float32)
          + jnp.dot(smax_sc[...], ax_ref[...],
                    preferred_element_type=jnp.float32)
          + jnp.dot(ssum, as_ref[...], preferred_element_type=jnp.float32)
          + mb1_ref[...]
      )
      gs = gs_ref[...]
      for kk in range(gs.shape[1]):
        zpre = zpre + gs[:, kk:kk + 1] * ast_ref[kk:kk + 1, :]
      z = jnp.maximum(zpre, 0.0)
      o_ref[...] = (
          jnp.dot(z, m2_ref[...], preferred_element_type=jnp.float32)
          + mb2_ref[...]
      )

  return pl.pallas_call(
      body,
      grid=(nb,),
      in_specs=[
          pl.BlockSpec((_NC, blk, d), lambda i: (0, i, 0)),
          pl.BlockSpec((blk, d), lambda i: (i, 0)),
          pl.BlockSpec((blk, 1), lambda i: (i, 0)),
          pl.BlockSpec((1, d), lambda i: (0, 0)),
          pl.BlockSpec((1, 1, blk), lambda i: (i, 0, 0)),
          pl.BlockSpec((blk, 1), lambda i: (i, 0)),
          pl.BlockSpec((g, 3), lambda i: (0, 0)),
          pl.BlockSpec((d, mh), lambda i: (0, 0)),
          pl.BlockSpec((d, mh), lambda i: (0, 0)),
          pl.BlockSpec((d, mh), lambda i: (0, 0)),
          pl.BlockSpec((3, mh), lambda i: (0, 0)),
          pl.BlockSpec((1, mh), lambda i: (0, 0)),
          pl.BlockSpec((mh, 1), lambda i: (0, 0)),
          pl.BlockSpec((1, 1), lambda i: (0, 0)),
      ],
      out_specs=pl.BlockSpec((g, 1), lambda i: (0, 0)),
      out_shape=jax.ShapeDtypeStruct((g, 1), jnp.float32),
      scratch_shapes=[
          pltpu.VMEM((g, d), jnp.float32),
          pltpu.VMEM((g, d), jnp.float32),
          pltpu.VMEM((g, 1), jnp.float32),
      ],
  )(parts, y, dis, b, batch_row, batch_col, gstats,
    a_mean, a_max, a_sum, a_st, mb1, m2, mb2)


def kernel(x, edge_index, batch, graph_stats, W1, b1, W2, b2, W3, b3,
           M1, mb1, M2, mb2):
  n, d_in = x.shape
  h = W1.shape[1]
  g = graph_stats.shape[0]
  src = edge_index[0].astype(jnp.int32)
  dst = edge_index[1].astype(jnp.int32)

  n_pad = ((n + 16 * _LANES - 1) // (16 * _LANES)) * (16 * _LANES)
  deg_parts = _sc_degree(dst, n_pad)
  deg_t = deg_parts.reshape(_NC * _NS, n_pad).T

  dis_full = _tc_dis(deg_t)
  dis = dis_full[:n]
  y = _tc_matmul_scale(x, W1, dis)

  parts = _sc_scatter(y, src, dst)
  y = _tc_layer(parts, y, dis, b1.reshape(1, h), W2)
  parts = _sc_scatter(y, src, dst)
  y = _tc_layer(parts, y, dis, b2.reshape(1, h), W3)
  parts = _sc_scatter(y, src, dst)

  a_mean = M1[:h]
  a_max = M1[h:2 * h]
  a_sum = M1[2 * h:3 * h]
  a_st = M1[3 * h:]
  out = _tc_final(
      parts, y, dis, b3.reshape(1, h),
      batch.astype(jnp.int32).reshape(-1, 1, 400),
      batch.astype(jnp.int32).reshape(n, 1),
      graph_stats,
      a_mean, a_max, a_sum, a_st,
      mb1.reshape(1, -1), M2, mb2.reshape(1, 1), g,
  )
  return jnp.squeeze(out)


# speedup vs baseline: 29.9999x; 1.0131x over previous
import dataclasses
import functools

import jax
import jax.numpy as jnp
from jax import lax
from jax.experimental import pallas as pl
from jax.experimental.pallas import tpu as pltpu
from jax.experimental.pallas import tpu_sc as plsc

_NC = 2
_NS = 16
_LANES = 16


def _sc_params():
  cp = pltpu.CompilerParams()
  if "needs_layout_passes" in pltpu.CompilerParams.__dataclass_fields__:
    cp = dataclasses.replace(cp, needs_layout_passes=False)
  return cp


def _sc_degree(dst, n_pad):
  e = dst.shape[0]
  nw = _NC * _NS
  per = e // nw
  assert per * nw == e and per % _LANES == 0
  mesh = plsc.VectorSubcoreMesh(core_axis_name="c", subcore_axis_name="s")

  @functools.partial(
      pl.kernel,
      out_type=jax.ShapeDtypeStruct((nw * n_pad,), jnp.float32),
      mesh=mesh,
      scratch_types=[
          pltpu.VMEM((n_pad,), jnp.float32),
          pltpu.VMEM((per,), jnp.int32),
      ],
      compiler_params=_sc_params(),
  )
  def k(dst_hbm, out_hbm, hist_v, idx_v):
    c = lax.axis_index("c")
    s = lax.axis_index("s")
    w = c * _NS + s

    @pl.loop(0, n_pad // _LANES)
    def _(i):
      hist_v[pl.ds(i * _LANES, _LANES)] = jnp.zeros((_LANES,), jnp.float32)

    pltpu.sync_copy(dst_hbm.at[pl.ds(w * per, per)], idx_v)
    ones = jnp.ones((_LANES,), jnp.float32)

    @pl.loop(0, per // _LANES, unroll=4)
    def _(i):
      idx = idx_v[pl.ds(i * _LANES, _LANES)]
      plsc.addupdate_scatter(hist_v, [idx], ones)

    pltpu.sync_copy(hist_v, out_hbm.at[pl.ds(w * n_pad, n_pad)])

  return k(dst)


def _sc_scatter(y, src, dst):
  n, d = y.shape
  e = src.shape[0]
  nw = _NC * _NS
  per = e // nw
  k_ch = 80
  nch = per // k_ch
  assert per * nw == e and nch * k_ch == per and (nch - 5) % 3 == 0
  rpt = (n // _NS) // 8 * 8
  rem = n - rpt * _NS
  assert rem % 8 == 0
  zr = 48
  nz = rpt // zr
  assert zr * nz == rpt and rem <= zr
  mesh = plsc.VectorSubcoreMesh(core_axis_name="c", subcore_axis_name="s")

  @functools.partial(
      pl.kernel,
      out_type=jax.ShapeDtypeStruct((_NC, n, d), jnp.float32),
      mesh=mesh,
      scratch_types=[
          pltpu.VMEM((per,), jnp.int32),
          pltpu.VMEM((k_ch,), jnp.int32),
          pltpu.VMEM((k_ch,), jnp.int32),
          pltpu.VMEM((k_ch,), jnp.int32),
          pltpu.VMEM((k_ch, d), jnp.float32),
          pltpu.VMEM((k_ch, d), jnp.float32),
          pltpu.VMEM((k_ch, d), jnp.float32),
          pltpu.VMEM((zr, d), jnp.float32),
          pltpu.VMEM_SHARED((n, d), jnp.float32),
          pltpu.SemaphoreType.DMA,
          pltpu.SemaphoreType.DMA,
          pltpu.SemaphoreType.DMA,
          pltpu.SemaphoreType.DMA,
          pltpu.SemaphoreType.DMA,
          pltpu.SemaphoreType.DMA,
          pltpu.SemaphoreType.DMA,
          pltpu.SemaphoreType.DMA,
          pltpu.SemaphoreType.DMA,
      ],
      compiler_params=_sc_params(),
  )
  def k(y_hbm, src_hbm, dst_hbm, out_hbm, src_v, dst_a, dst_b, dst_c,
        buf_a, buf_b, buf_c, zero_v, acc_sh,
        gsem_a, gsem_b, gsem_c, ssem_a, ssem_b, ssem_c,
        dsem_a, dsem_b, dsem_c):
    c = lax.axis_index("c")
    s = lax.axis_index("s")
    w = c * _NS + s
    bufs = (buf_a, buf_b, buf_c)
    dsts = (dst_a, dst_b, dst_c)
    gsems = (gsem_a, gsem_b, gsem_c)
    ssems = (ssem_a, ssem_b, ssem_c)
    dsems = (dsem_a, dsem_b, dsem_c)

    @pl.loop(0, zr)
    def _(i):
      for j in range(d // _LANES):
        zero_v[i, pl.ds(j * _LANES, _LANES)] = jnp.zeros((_LANES,), jnp.float32)

    for t in range(nz):
      pltpu.sync_copy(zero_v, acc_sh.at[pl.ds(s * rpt + t * zr, zr)])

    @pl.when(s == 0)
    def _():
      pltpu.sync_copy(zero_v.at[pl.ds(0, rem)], acc_sh.at[pl.ds(_NS * rpt, rem)])

    base0 = w * per
    pltpu.sync_copy(src_hbm.at[pl.ds(base0, per)], src_v)
    plsc.subcore_barrier()

    def g_start(i, p):
      pltpu.async_copy(y_hbm.at[src_v.at[pl.ds(i * k_ch, k_ch)]],
                       bufs[p], gsems[p])

    def g_wait(p):
      pltpu.make_async_copy(y_hbm.at[src_v.at[pl.ds(0, k_ch)]],
                            bufs[p], gsems[p]).wait()

    def d_start(i, p):
      pltpu.async_copy(dst_hbm.at[pl.ds(base0 + i * k_ch, k_ch)],
                       dsts[p], dsems[p])

    def d_wait(p):
      pltpu.make_async_copy(dst_hbm.at[pl.ds(base0, k_ch)],
                            dsts[p], dsems[p]).wait()

    def s_start(p):
      pltpu.async_copy(bufs[p], acc_sh.at[dsts[p]], ssems[p], add=True)

    def s_wait(p):
      pltpu.make_async_copy(bufs[p], acc_sh.at[dsts[p]], ssems[p]).wait()

    d_start(0, 0)
    g_start(0, 0)
    d_start(1, 1)
    g_start(1, 1)

    def step(i, p, first_round):
      g_wait(p)
      d_wait(p)
      s_start(p)
      q = (p + 2) % 3
      if not (first_round and q == 2):
        s_wait(q)
      d_start(i + 2, q)
      g_start(i + 2, q)

    step(0, 0, True)
    step(1, 1, True)
    step(2, 2, False)

    @pl.loop(0, (nch - 5) // 3)
    def _(j):
      base = 3 + 3 * j
      step(base + 0, 0, False)
      step(base + 1, 1, False)
      step(base + 2, 2, False)

    g_wait(0)
    d_wait(0)
    s_start(0)
    g_wait(1)
    d_wait(1)
    s_start(1)
    s_wait(0)
    s_wait(1)
    s_wait(2)

    plsc.subcore_barrier()
    pltpu.sync_copy(
        acc_sh.at[pl.ds(s * rpt, rpt)],
        out_hbm.at[c, pl.ds(s * rpt, rpt)],
    )

    @pl.when(s == 0)
    def _():
      pltpu.sync_copy(
          acc_sh.at[pl.ds(_NS * rpt, rem)],
          out_hbm.at[c, pl.ds(_NS * rpt, rem)],
      )

  return k(y, src, dst)


_BLK = 1000


def _tc_matmul(h, w):
  n, d_in = h.shape
  d_out = w.shape[1]

  def body(h_ref, w_ref, y_ref):
    y_ref[...] = jnp.dot(h_ref[...], w_ref[...],
                         preferred_element_type=jnp.float32)

  return pl.pallas_call(
      body,
      grid=(n // _BLK,),
      in_specs=[
          pl.BlockSpec((_BLK, d_in), lambda i: (i, 0)),
          pl.BlockSpec((d_in, d_out), lambda i: (0, 0)),
      ],
      out_specs=pl.BlockSpec((_BLK, d_out), lambda i: (i, 0)),
      out_shape=jax.ShapeDtypeStruct((n, d_out), jnp.float32),
  )(h, w)


def _tc_dis_scale(deg_t, mm):
  n, d = mm.shape
  nw = deg_t.shape[1]

  def body(deg_ref, mm_ref, dis_ref, y_ref):
    deg = jnp.sum(deg_ref[...], axis=1, keepdims=True) + 1.0
    dis = lax.rsqrt(deg)
    dis_ref[...] = dis
    y_ref[...] = mm_ref[...] * dis

  return pl.pallas_call(
      body,
      grid=(n // _BLK,),
      in_specs=[
          pl.BlockSpec((_BLK, nw), lambda i: (i, 0)),
          pl.BlockSpec((_BLK, d), lambda i: (i, 0)),
      ],
      out_specs=(
          pl.BlockSpec((_BLK, 1), lambda i: (i, 0)),
          pl.BlockSpec((_BLK, d), lambda i: (i, 0)),
      ),
      out_shape=(
          jax.ShapeDtypeStruct((n, 1), jnp.float32),
          jax.ShapeDtypeStruct((n, d), jnp.float32),
      ),
  )(deg_t, mm)


def _tc_layer(parts, y, dis, b, w_next):
  n, d = y.shape

  def body(p_ref, y_ref, d_ref, b_ref, w_ref, o_ref):
    t = (p_ref[0] + p_ref[1] + y_ref[...]) * d_ref[...] + b_ref[...]
    z = jnp.maximum(t, 0.0)
    o_ref[...] = (
        jnp.dot(z, w_ref[...], preferred_element_type=jnp.float32) * d_ref[...]
    )

  return pl.pallas_call(
      body,
      grid=(n // _BLK,),
      in_specs=[
          pl.BlockSpec((_NC, _BLK, d), lambda i: (0, i, 0)),
          pl.BlockSpec((_BLK, d), lambda i: (i, 0)),
          pl.BlockSpec((_BLK, 1), lambda i: (i, 0)),
          pl.BlockSpec((1, d), lambda i: (0, 0)),
          pl.BlockSpec((d, d), lambda i: (0, 0)),
      ],
      out_specs=pl.BlockSpec((_BLK, d), lambda i: (i, 0)),
      out_shape=jax.ShapeDtypeStruct((n, d), jnp.float32),
  )(parts, y, dis, b, w_next)


def _tc_final(parts, y, dis, b, batch_row, batch_col, gstats,
              a_mean, a_max, a_sum, a_st, mb1, m2, mb2, num_graphs):
  n, d = y.shape
  g = num_graphs
  mh = m2.shape[0]
  blk = 400
  nb = n // blk
  assert nb * blk == n

  def body(p_ref, y_ref, d_ref, b_ref, br_ref, bc_ref, gs_ref,
           am_ref, ax_ref, as_ref, ast_ref, mb1_ref, m2_ref, mb2_ref, o_ref,
           ssum_sc, smax_sc, cnt_sc):
    i = pl.program_id(0)

    @pl.when(i == 0)
    def _():
      ssum_sc[...] = jnp.zeros_like(ssum_sc)
      smax_sc[...] = jnp.full_like(smax_sc, -jnp.inf)
      cnt_sc[...] = jnp.zeros_like(cnt_sc)

    t = (p_ref[0] + p_ref[1] + y_ref[...]) * d_ref[...] + b_ref[...]
    h = jnp.maximum(t, 0.0)

    gid = lax.broadcasted_iota(jnp.int32, (g, 1), 0)
    oh = (br_ref[0] == gid).astype(jnp.float32)
    ssum_sc[...] += jnp.dot(oh, h, preferred_element_type=jnp.float32)
    cnt_sc[...] += jnp.sum(oh, axis=1, keepdims=True)

    bc = bc_ref[...]
    neg = jnp.float32(-jnp.inf)
    mx_rows = []
    for gg in range(g):
      m = jnp.where(bc == gg, h, neg)
      mx_rows.append(jnp.max(m, axis=0, keepdims=True))
    smax_sc[...] = jnp.maximum(smax_sc[...], jnp.concatenate(mx_rows, axis=0))

    @pl.when(i == nb - 1)
    def _():
      ssum = ssum_sc[...]
      mean = ssum / jnp.maximum(cnt_sc[...], 1.0)
      zpre = (
          jnp.dot(mean, am_ref[...], preferred_element_type=jnp.float32)
          + jnp.dot(smax_sc[...], ax_ref[...],
                    preferred_element_type=jnp.float32)
          + jnp.dot(ssum, as_ref[...], preferred_element_type=jnp.float32)
          + mb1_ref[...]
      )
      gs = gs_ref[...]
      for kk in range(gs.shape[1]):
        zpre = zpre + gs[:, kk:kk + 1] * ast_ref[kk:kk + 1, :]
      z = jnp.maximum(zpre, 0.0)
      o_ref[...] = (
          jnp.dot(z, m2_ref[...], preferred_element_type=jnp.float32)
          + mb2_ref[...]
      )

  return pl.pallas_call(
      body,
      grid=(nb,),
      in_specs=[
          pl.BlockSpec((_NC, blk, d), lambda i: (0, i, 0)),
          pl.BlockSpec((blk, d), lambda i: (i, 0)),
          pl.BlockSpec((blk, 1), lambda i: (i, 0)),
          pl.BlockSpec((1, d), lambda i: (0, 0)),
          pl.BlockSpec((1, 1, blk), lambda i: (i, 0, 0)),
          pl.BlockSpec((blk, 1), lambda i: (i, 0)),
          pl.BlockSpec((g, 3), lambda i: (0, 0)),
          pl.BlockSpec((d, mh), lambda i: (0, 0)),
          pl.BlockSpec((d, mh), lambda i: (0, 0)),
          pl.BlockSpec((d, mh), lambda i: (0, 0)),
          pl.BlockSpec((3, mh), lambda i: (0, 0)),
          pl.BlockSpec((1, mh), lambda i: (0, 0)),
          pl.BlockSpec((mh, 1), lambda i: (0, 0)),
          pl.BlockSpec((1, 1), lambda i: (0, 0)),
      ],
      out_specs=pl.BlockSpec((g, 1), lambda i: (0, 0)),
      out_shape=jax.ShapeDtypeStruct((g, 1), jnp.float32),
      scratch_shapes=[
          pltpu.VMEM((g, d), jnp.float32),
          pltpu.VMEM((g, d), jnp.float32),
          pltpu.VMEM((g, 1), jnp.float32),
      ],
  )(parts, y, dis, b, batch_row, batch_col, gstats,
    a_mean, a_max, a_sum, a_st, mb1, m2, mb2)


def kernel(x, edge_index, batch, graph_stats, W1, b1, W2, b2, W3, b3,
           M1, mb1, M2, mb2):
  n, d_in = x.shape
  h = W1.shape[1]
  g = graph_stats.shape[0]
  src = edge_index[0].astype(jnp.int32)
  dst = edge_index[1].astype(jnp.int32)

  n_pad = ((n + 16 * _LANES - 1) // (16 * _LANES)) * (16 * _LANES)
  deg_parts = _sc_degree(dst, n_pad)
  deg_t = deg_parts.reshape(_NC * _NS, n_pad).T

  mm = _tc_matmul(x, W1)
  dis, y = _tc_dis_scale(deg_t[:n], mm)

  parts = _sc_scatter(y, src, dst)
  y = _tc_layer(parts, y, dis, b1.reshape(1, h), W2)
  parts = _sc_scatter(y, src, dst)
  y = _tc_layer(parts, y, dis, b2.reshape(1, h), W3)
  parts = _sc_scatter(y, src, dst)

  a_mean = M1[:h]
  a_max = M1[h:2 * h]
  a_sum = M1[2 * h:3 * h]
  a_st = M1[3 * h:]
  out = _tc_final(
      parts, y, dis, b3.reshape(1, h),
      batch.astype(jnp.int32).reshape(-1, 1, 400),
      batch.astype(jnp.int32).reshape(n, 1),
      graph_stats,
      a_mean, a_max, a_sum, a_st,
      mb1.reshape(1, -1), M2, mb2.reshape(1, 1), g,
  )
  return jnp.squeeze(out)


# speedup vs baseline: 30.8212x; 1.0274x over previous
import dataclasses
import functools

import jax
import jax.numpy as jnp
from jax import lax
from jax.experimental import pallas as pl
from jax.experimental.pallas import tpu as pltpu
from jax.experimental.pallas import tpu_sc as plsc

_NC = 2
_NS = 16
_LANES = 16


def _sc_params():
  cp = pltpu.CompilerParams()
  if "needs_layout_passes" in pltpu.CompilerParams.__dataclass_fields__:
    cp = dataclasses.replace(cp, needs_layout_passes=False)
  return cp


def _sc_degree(dst, n_pad):
  e = dst.shape[0]
  nw = _NC * _NS
  per = e // nw
  col = n_pad // _NS
  assert per * nw == e and per % _LANES == 0 and col % 128 == 0
  mesh = plsc.VectorSubcoreMesh(core_axis_name="c", subcore_axis_name="s")

  @functools.partial(
      pl.kernel,
      out_type=jax.ShapeDtypeStruct((_NC * n_pad,), jnp.float32),
      mesh=mesh,
      scratch_types=[
          pltpu.VMEM((n_pad,), jnp.float32),
          pltpu.VMEM((per,), jnp.int32),
          pltpu.VMEM((_NS, col), jnp.float32),
          pltpu.VMEM_SHARED((_NS, n_pad), jnp.float32),
      ],
      compiler_params=_sc_params(),
  )
  def k(dst_hbm, out_hbm, hist_v, idx_v, comb_v, sh):
    c = lax.axis_index("c")
    s = lax.axis_index("s")
    w = c * _NS + s

    @pl.loop(0, n_pad // _LANES)
    def _(i):
      hist_v[pl.ds(i * _LANES, _LANES)] = jnp.zeros((_LANES,), jnp.float32)

    pltpu.sync_copy(dst_hbm.at[pl.ds(w * per, per)], idx_v)
    ones = jnp.ones((_LANES,), jnp.float32)

    @pl.loop(0, per // _LANES, unroll=4)
    def _(i):
      idx = idx_v[pl.ds(i * _LANES, _LANES)]
      plsc.addupdate_scatter(hist_v, [idx], ones)

    pltpu.sync_copy(hist_v, sh.at[s])
    plsc.subcore_barrier()
    pltpu.sync_copy(sh.at[:, pl.ds(s * col, col)], comb_v)

    @pl.loop(0, col // _LANES, unroll=4)
    def _(j):
      acc = comb_v[0, pl.ds(j * _LANES, _LANES)]
      for r in range(1, _NS):
        acc = acc + comb_v[r, pl.ds(j * _LANES, _LANES)]
      comb_v[0, pl.ds(j * _LANES, _LANES)] = acc

    pltpu.sync_copy(comb_v.at[0],
                    out_hbm.at[pl.ds(c * n_pad + s * col, col)])

  return k(dst)


def _sc_scatter(y, src, dst):
  n, d = y.shape
  e = src.shape[0]
  nw = _NC * _NS
  per = e // nw
  k_ch = 80
  nch = per // k_ch
  assert per * nw == e and nch * k_ch == per and (nch - 5) % 3 == 0
  rpt = (n // _NS) // 8 * 8
  rem = n - rpt * _NS
  assert rem % 8 == 0
  zr = 48
  nz = rpt // zr
  assert zr * nz == rpt and rem <= zr
  mesh = plsc.VectorSubcoreMesh(core_axis_name="c", subcore_axis_name="s")

  @functools.partial(
      pl.kernel,
      out_type=jax.ShapeDtypeStruct((_NC, n, d), jnp.float32),
      mesh=mesh,
      scratch_types=[
          pltpu.VMEM((per,), jnp.int32),
          pltpu.VMEM((k_ch,), jnp.int32),
          pltpu.VMEM((k_ch,), jnp.int32),
          pltpu.VMEM((k_ch,), jnp.int32),
          pltpu.VMEM((k_ch, d), jnp.float32),
          pltpu.VMEM((k_ch, d), jnp.float32),
          pltpu.VMEM((k_ch, d), jnp.float32),
          pltpu.VMEM((zr, d), jnp.float32),
          pltpu.VMEM_SHARED((n, d), jnp.float32),
          pltpu.SemaphoreType.DMA,
          pltpu.SemaphoreType.DMA,
          pltpu.SemaphoreType.DMA,
          pltpu.SemaphoreType.DMA,
          pltpu.SemaphoreType.DMA,
          pltpu.SemaphoreType.DMA,
          pltpu.SemaphoreType.DMA,
          pltpu.SemaphoreType.DMA,
          pltpu.SemaphoreType.DMA,
      ],
      compiler_params=_sc_params(),
  )
  def k(y_hbm, src_hbm, dst_hbm, out_hbm, src_v, dst_a, dst_b, dst_c,
        buf_a, buf_b, buf_c, zero_v, acc_sh,
        gsem_a, gsem_b, gsem_c, ssem_a, ssem_b, ssem_c,
        dsem_a, dsem_b, dsem_c):
    c = lax.axis_index("c")
    s = lax.axis_index("s")
    w = c * _NS + s
    bufs = (buf_a, buf_b, buf_c)
    dsts = (dst_a, dst_b, dst_c)
    gsems = (gsem_a, gsem_b, gsem_c)
    ssems = (ssem_a, ssem_b, ssem_c)
    dsems = (dsem_a, dsem_b, dsem_c)

    @pl.loop(0, zr)
    def _(i):
      for j in range(d // _LANES):
        zero_v[i, pl.ds(j * _LANES, _LANES)] = jnp.zeros((_LANES,), jnp.float32)

    for t in range(nz):
      pltpu.sync_copy(zero_v, acc_sh.at[pl.ds(s * rpt + t * zr, zr)])

    @pl.when(s == 0)
    def _():
      pltpu.sync_copy(zero_v.at[pl.ds(0, rem)], acc_sh.at[pl.ds(_NS * rpt, rem)])

    base0 = w * per
    pltpu.sync_copy(src_hbm.at[pl.ds(base0, per)], src_v)
    plsc.subcore_barrier()

    def g_start(i, p):
      pltpu.async_copy(y_hbm.at[src_v.at[pl.ds(i * k_ch, k_ch)]],
                       bufs[p], gsems[p])

    def g_wait(p):
      pltpu.make_async_copy(y_hbm.at[src_v.at[pl.ds(0, k_ch)]],
                            bufs[p], gsems[p]).wait()

    def d_start(i, p):
      pltpu.async_copy(dst_hbm.at[pl.ds(base0 + i * k_ch, k_ch)],
                       dsts[p], dsems[p])

    def d_wait(p):
      pltpu.make_async_copy(dst_hbm.at[pl.ds(base0, k_ch)],
                            dsts[p], dsems[p]).wait()

    def s_start(p):
      pltpu.async_copy(bufs[p], acc_sh.at[dsts[p]], ssems[p], add=True)

    def s_wait(p):
      pltpu.make_async_copy(bufs[p], acc_sh.at[dsts[p]], ssems[p]).wait()

    d_start(0, 0)
    g_start(0, 0)
    d_start(1, 1)
    g_start(1, 1)

    def step(i, p, first_round):
      g_wait(p)
      d_wait(p)
      s_start(p)
      q = (p + 2) % 3
      if not (first_round and q == 2):
        s_wait(q)
      d_start(i + 2, q)
      g_start(i + 2, q)

    step(0, 0, True)
    step(1, 1, True)
    step(2, 2, False)

    @pl.loop(0, (nch - 5) // 3)
    def _(j):
      base = 3 + 3 * j
      step(base + 0, 0, False)
      step(base + 1, 1, False)
      step(base + 2, 2, False)

    g_wait(0)
    d_wait(0)
    s_start(0)
    g_wait(1)
    d_wait(1)
    s_start(1)
    s_wait(0)
    s_wait(1)
    s_wait(2)

    plsc.subcore_barrier()
    pltpu.sync_copy(
        acc_sh.at[pl.ds(s * rpt, rpt)],
        out_hbm.at[c, pl.ds(s * rpt, rpt)],
    )

    @pl.when(s == 0)
    def _():
      pltpu.sync_copy(
          acc_sh.at[pl.ds(_NS * rpt, rem)],
          out_hbm.at[c, pl.ds(_NS * rpt, rem)],
      )

  return k(y, src, dst)


_BLK = 2000


def _tc_matmul(h, w):
  n, d_in = h.shape
  d_out = w.shape[1]

  def body(h_ref, w_ref, y_ref):
    y_ref[...] = jnp.dot(h_ref[...], w_ref[...],
                         preferred_element_type=jnp.float32)

  return pl.pallas_call(
      body,
      grid=(n // _BLK,),
      in_specs=[
          pl.BlockSpec((_BLK, d_in), lambda i: (i, 0)),
          pl.BlockSpec((d_in, d_out), lambda i: (0, 0)),
      ],
      out_specs=pl.BlockSpec((_BLK, d_out), lambda i: (i, 0)),
      out_shape=jax.ShapeDtypeStruct((n, d_out), jnp.float32),
  )(h, w)


def _tc_dis_scale(deg_t, mm):
  n, d = mm.shape
  nw = deg_t.shape[1]

  def body(deg_ref, mm_ref, dis_ref, y_ref):
    deg = jnp.sum(deg_ref[...], axis=1, keepdims=True) + 1.0
    dis = lax.rsqrt(deg)
    dis_ref[...] = dis
    y_ref[...] = mm_ref[...] * dis

  return pl.pallas_call(
      body,
      grid=(n // _BLK,),
      in_specs=[
          pl.BlockSpec((_BLK, nw), lambda i: (i, 0)),
          pl.BlockSpec((_BLK, d), lambda i: (i, 0)),
      ],
      out_specs=(
          pl.BlockSpec((_BLK, 1), lambda i: (i, 0)),
          pl.BlockSpec((_BLK, d), lambda i: (i, 0)),
      ),
      out_shape=(
          jax.ShapeDtypeStruct((n, 1), jnp.float32),
          jax.ShapeDtypeStruct((n, d), jnp.float32),
      ),
  )(deg_t, mm)


def _tc_layer(parts, y, dis, b, w_next):
  n, d = y.shape

  def body(p_ref, y_ref, d_ref, b_ref, w_ref, o_ref):
    t = (p_ref[0] + p_ref[1] + y_ref[...]) * d_ref[...] + b_ref[...]
    z = jnp.maximum(t, 0.0)
    o_ref[...] = (
        jnp.dot(z, w_ref[...], preferred_element_type=jnp.float32) * d_ref[...]
    )

  return pl.pallas_call(
      body,
      grid=(n // _BLK,),
      in_specs=[
          pl.BlockSpec((_NC, _BLK, d), lambda i: (0, i, 0)),
          pl.BlockSpec((_BLK, d), lambda i: (i, 0)),
          pl.BlockSpec((_BLK, 1), lambda i: (i, 0)),
          pl.BlockSpec((1, d), lambda i: (0, 0)),
          pl.BlockSpec((d, d), lambda i: (0, 0)),
      ],
      out_specs=pl.BlockSpec((_BLK, d), lambda i: (i, 0)),
      out_shape=jax.ShapeDtypeStruct((n, d), jnp.float32),
  )(parts, y, dis, b, w_next)


def _tc_final(parts, y, dis, b, batch_row, batch_col, gstats,
              a_mean, a_max, a_sum, a_st, mb1, m2, mb2, num_graphs):
  n, d = y.shape
  g = num_graphs
  mh = m2.shape[0]
  blk = 400
  nb = n // blk
  assert nb * blk == n

  def body(p_ref, y_ref, d_ref, b_ref, br_ref, bc_ref, gs_ref,
           am_ref, ax_ref, as_ref, ast_ref, mb1_ref, m2_ref, mb2_ref, o_ref,
           ssum_sc, smax_sc, cnt_sc):
    i = pl.program_id(0)

    @pl.when(i == 0)
    def _():
      ssum_sc[...] = jnp.zeros_like(ssum_sc)
      smax_sc[...] = jnp.full_like(smax_sc, -jnp.inf)
      cnt_sc[...] = jnp.zeros_like(cnt_sc)

    t = (p_ref[0] + p_ref[1] + y_ref[...]) * d_ref[...] + b_ref[...]
    h = jnp.maximum(t, 0.0)

    gid = lax.broadcasted_iota(jnp.int32, (g, 1), 0)
    oh = (br_ref[0] == gid).astype(jnp.float32)
    ssum_sc[...] += jnp.dot(oh, h, preferred_element_type=jnp.float32)
    cnt_sc[...] += jnp.sum(oh, axis=1, keepdims=True)

    bc = bc_ref[...]
    neg = jnp.float32(-jnp.inf)
    mx_rows = []
    for gg in range(g):
      m = jnp.where(bc == gg, h, neg)
      mx_rows.append(jnp.max(m, axis=0, keepdims=True))
    smax_sc[...] = jnp.maximum(smax_sc[...], jnp.concatenate(mx_rows, axis=0))

    @pl.when(i == nb - 1)
    def _():
      ssum = ssum_sc[...]
      mean = ssum / jnp.maximum(cnt_sc[...], 1.0)
      zpre = (
          jnp.dot(mean, am_ref[...], preferred_element_type=jnp.float32)
          + jnp.dot(smax_sc[...], ax_ref[...],
                    preferred_element_type=jnp.float32)
          + jnp.dot(ssum, as_ref[...], preferred_element_type=jnp.float32)
          + mb1_ref[...]
      )
      gs = gs_ref[...]
      for kk in range(gs.shape[1]):
        zpre = zpre + gs[:, kk:kk + 1] * ast_ref[kk:kk + 1, :]
      z = jnp.maximum(zpre, 0.0)
      o_ref[...] = (
          jnp.dot(z, m2_ref[...], preferred_element_type=jnp.float32)
          + mb2_ref[...]
      )

  return pl.pallas_call(
      body,
      grid=(nb,),
      in_specs=[
          pl.BlockSpec((_NC, blk, d), lambda i: (0, i, 0)),
          pl.BlockSpec((blk, d), lambda i: (i, 0)),
          pl.BlockSpec((blk, 1), lambda i: (i, 0)),
          pl.BlockSpec((1, d), lambda i: (0, 0)),
          pl.BlockSpec((1, 1, blk), lambda i: (i, 0, 0)),
          pl.BlockSpec((blk, 1), lambda i: (i, 0)),
          pl.BlockSpec((g, 3), lambda i: (0, 0)),
          pl.BlockSpec((d, mh), lambda i: (0, 0)),
          pl.BlockSpec((d, mh), lambda i: (0, 0)),
          pl.BlockSpec((d, mh), lambda i: (0, 0)),
          pl.BlockSpec((3, mh), lambda i: (0, 0)),
          pl.BlockSpec((1, mh), lambda i: (0, 0)),
          pl.BlockSpec((mh, 1), lambda i: (0, 0)),
          pl.BlockSpec((1, 1), lambda i: (0, 0)),
      ],
      out_specs=pl.BlockSpec((g, 1), lambda i: (0, 0)),
      out_shape=jax.ShapeDtypeStruct((g, 1), jnp.float32),
      scratch_shapes=[
          pltpu.VMEM((g, d), jnp.float32),
          pltpu.VMEM((g, d), jnp.float32),
          pltpu.VMEM((g, 1), jnp.float32),
      ],
  )(parts, y, dis, b, batch_row, batch_col, gstats,
    a_mean, a_max, a_sum, a_st, mb1, m2, mb2)


def kernel(x, edge_index, batch, graph_stats, W1, b1, W2, b2, W3, b3,
           M1, mb1, M2, mb2):
  n, d_in = x.shape
  h = W1.shape[1]
  g = graph_stats.shape[0]
  src = edge_index[0].astype(jnp.int32)
  dst = edge_index[1].astype(jnp.int32)

  n_pad = ((n + 16 * _LANES - 1) // (16 * _LANES)) * (16 * _LANES)
  deg_parts = _sc_degree(dst, n_pad)
  deg_t = deg_parts.reshape(_NC, n_pad).T

  mm = _tc_matmul(x, W1)
  dis, y = _tc_dis_scale(deg_t, mm)

  parts = _sc_scatter(y, src, dst)
  y = _tc_layer(parts, y, dis, b1.reshape(1, h), W2)
  parts = _sc_scatter(y, src, dst)
  y = _tc_layer(parts, y, dis, b2.reshape(1, h), W3)
  parts = _sc_scatter(y, src, dst)

  a_mean = M1[:h]
  a_max = M1[h:2 * h]
  a_sum = M1[2 * h:3 * h]
  a_st = M1[3 * h:]
  out = _tc_final(
      parts, y, dis, b3.reshape(1, h),
      batch.astype(jnp.int32).reshape(-1, 1, 400),
      batch.astype(jnp.int32).reshape(n, 1),
      graph_stats,
      a_mean, a_max, a_sum, a_st,
      mb1.reshape(1, -1), M2, mb2.reshape(1, 1), g,
  )
  return jnp.squeeze(out)


# speedup vs baseline: 31.2305x; 1.0133x over previous
import dataclasses
import functools

import jax
import jax.numpy as jnp
from jax import lax
from jax.experimental import pallas as pl
from jax.experimental.pallas import tpu as pltpu
from jax.experimental.pallas import tpu_sc as plsc

_NC = 2
_NS = 16
_LANES = 16


def _sc_params():
  cp = pltpu.CompilerParams()
  if "needs_layout_passes" in pltpu.CompilerParams.__dataclass_fields__:
    cp = dataclasses.replace(cp, needs_layout_passes=False)
  return cp


def _sc_degree(dst, n_pad):
  e = dst.shape[0]
  nw = _NC * _NS
  per = e // nw
  col = n_pad // _NS
  assert per * nw == e and per % _LANES == 0 and col % 128 == 0
  mesh = plsc.VectorSubcoreMesh(core_axis_name="c", subcore_axis_name="s")

  @functools.partial(
      pl.kernel,
      out_type=jax.ShapeDtypeStruct((_NC * n_pad,), jnp.float32),
      mesh=mesh,
      scratch_types=[
          pltpu.VMEM((n_pad,), jnp.float32),
          pltpu.VMEM((per,), jnp.int32),
          pltpu.VMEM((_NS, col), jnp.float32),
          pltpu.VMEM_SHARED((_NS, n_pad), jnp.float32),
      ],
      compiler_params=_sc_params(),
  )
  def k(dst_hbm, out_hbm, hist_v, idx_v, comb_v, sh):
    c = lax.axis_index("c")
    s = lax.axis_index("s")
    w = c * _NS + s

    @pl.loop(0, n_pad // _LANES)
    def _(i):
      hist_v[pl.ds(i * _LANES, _LANES)] = jnp.zeros((_LANES,), jnp.float32)

    pltpu.sync_copy(dst_hbm.at[pl.ds(w * per, per)], idx_v)
    ones = jnp.ones((_LANES,), jnp.float32)

    @pl.loop(0, per // _LANES, unroll=4)
    def _(i):
      idx = idx_v[pl.ds(i * _LANES, _LANES)]
      plsc.addupdate_scatter(hist_v, [idx], ones)

    pltpu.sync_copy(hist_v, sh.at[s])
    plsc.subcore_barrier()
    pltpu.sync_copy(sh.at[:, pl.ds(s * col, col)], comb_v)

    @pl.loop(0, col // _LANES, unroll=4)
    def _(j):
      acc = comb_v[0, pl.ds(j * _LANES, _LANES)]
      for r in range(1, _NS):
        acc = acc + comb_v[r, pl.ds(j * _LANES, _LANES)]
      comb_v[0, pl.ds(j * _LANES, _LANES)] = acc

    pltpu.sync_copy(comb_v.at[0],
                    out_hbm.at[pl.ds(c * n_pad + s * col, col)])

  return k(dst)


def _sc_scatter(y, src, dst):
  n, d = y.shape
  e = src.shape[0]
  nw = _NC * _NS
  per = e // nw
  k_ch = 80
  nch = per // k_ch
  assert per * nw == e and nch * k_ch == per and (nch - 5) % 3 == 0
  rpt = (n // _NS) // 8 * 8
  rem = n - rpt * _NS
  assert rem % 8 == 0
  zr = 48
  nz = rpt // zr
  assert zr * nz == rpt and rem <= zr
  mesh = plsc.VectorSubcoreMesh(core_axis_name="c", subcore_axis_name="s")

  @functools.partial(
      pl.kernel,
      out_type=jax.ShapeDtypeStruct((_NC, n, d), jnp.float32),
      mesh=mesh,
      scratch_types=[
          pltpu.VMEM((per,), jnp.int32),
          pltpu.VMEM((k_ch,), jnp.int32),
          pltpu.VMEM((k_ch,), jnp.int32),
          pltpu.VMEM((k_ch,), jnp.int32),
          pltpu.VMEM((k_ch, d), jnp.float32),
          pltpu.VMEM((k_ch, d), jnp.float32),
          pltpu.VMEM((k_ch, d), jnp.float32),
          pltpu.VMEM((zr, d), jnp.float32),
          pltpu.VMEM_SHARED((n, d), jnp.float32),
          pltpu.SemaphoreType.DMA,
          pltpu.SemaphoreType.DMA,
          pltpu.SemaphoreType.DMA,
          pltpu.SemaphoreType.DMA,
          pltpu.SemaphoreType.DMA,
          pltpu.SemaphoreType.DMA,
          pltpu.SemaphoreType.DMA,
          pltpu.SemaphoreType.DMA,
          pltpu.SemaphoreType.DMA,
          pltpu.SemaphoreType.DMA,
      ],
      compiler_params=_sc_params(),
  )
  def k(y_hbm, src_hbm, dst_hbm, out_hbm, src_v, dst_a, dst_b, dst_c,
        buf_a, buf_b, buf_c, zero_v, acc_sh,
        gsem_a, gsem_b, gsem_c, ssem_a, ssem_b, ssem_c,
        dsem_a, dsem_b, dsem_c, zsem):
    c = lax.axis_index("c")
    s = lax.axis_index("s")
    w = c * _NS + s
    bufs = (buf_a, buf_b, buf_c)
    dsts = (dst_a, dst_b, dst_c)
    gsems = (gsem_a, gsem_b, gsem_c)
    ssems = (ssem_a, ssem_b, ssem_c)
    dsems = (dsem_a, dsem_b, dsem_c)

    base0 = w * per
    pltpu.sync_copy(src_hbm.at[pl.ds(base0, per)], src_v)

    def g_start(i, p):
      pltpu.async_copy(y_hbm.at[src_v.at[pl.ds(i * k_ch, k_ch)]],
                       bufs[p], gsems[p])

    def g_wait(p):
      pltpu.make_async_copy(y_hbm.at[src_v.at[pl.ds(0, k_ch)]],
                            bufs[p], gsems[p]).wait()

    def d_start(i, p):
      pltpu.async_copy(dst_hbm.at[pl.ds(base0 + i * k_ch, k_ch)],
                       dsts[p], dsems[p])

    def d_wait(p):
      pltpu.make_async_copy(dst_hbm.at[pl.ds(base0, k_ch)],
                            dsts[p], dsems[p]).wait()

    def s_start(p):
      pltpu.async_copy(bufs[p], acc_sh.at[dsts[p]], ssems[p], add=True)

    def s_wait(p):
      pltpu.make_async_copy(bufs[p], acc_sh.at[dsts[p]], ssems[p]).wait()

    d_start(0, 0)
    g_start(0, 0)
    d_start(1, 1)
    g_start(1, 1)

    @pl.loop(0, zr)
    def _(i):
      for j in range(d // _LANES):
        zero_v[i, pl.ds(j * _LANES, _LANES)] = jnp.zeros((_LANES,), jnp.float32)

    for t in range(nz):
      pltpu.async_copy(zero_v, acc_sh.at[pl.ds(s * rpt + t * zr, zr)], zsem)
    for t in range(nz):
      pltpu.make_async_copy(zero_v, acc_sh.at[pl.ds(s * rpt, zr)], zsem).wait()

    @pl.when(s == 0)
    def _():
      pltpu.async_copy(zero_v.at[pl.ds(0, rem)],
                       acc_sh.at[pl.ds(_NS * rpt, rem)], zsem)
      pltpu.make_async_copy(zero_v.at[pl.ds(0, rem)],
                            acc_sh.at[pl.ds(_NS * rpt, rem)], zsem).wait()

    plsc.subcore_barrier()

    def step(i, p, first_round):
      g_wait(p)
      d_wait(p)
      s_start(p)
      q = (p + 2) % 3
      if not (first_round and q == 2):
        s_wait(q)
      d_start(i + 2, q)
      g_start(i + 2, q)

    step(0, 0, True)
    step(1, 1, True)
    step(2, 2, False)

    @pl.loop(0, (nch - 5) // 3)
    def _(j):
      base = 3 + 3 * j
      step(base + 0, 0, False)
      step(base + 1, 1, False)
      step(base + 2, 2, False)

    g_wait(0)
    d_wait(0)
    s_start(0)
    g_wait(1)
    d_wait(1)
    s_start(1)
    s_wait(0)
    s_wait(1)
    s_wait(2)

    plsc.subcore_barrier()
    pltpu.sync_copy(
        acc_sh.at[pl.ds(s * rpt, rpt)],
        out_hbm.at[c, pl.ds(s * rpt, rpt)],
    )

    @pl.when(s == 0)
    def _():
      pltpu.sync_copy(
          acc_sh.at[pl.ds(_NS * rpt, rem)],
          out_hbm.at[c, pl.ds(_NS * rpt, rem)],
      )

  return k(y, src, dst)


_BLK = 2000


def _tc_matmul(h, w):
  n, d_in = h.shape
  d_out = w.shape[1]

  def body(h_ref, w_ref, y_ref):
    y_ref[...] = jnp.dot(h_ref[...], w_ref[...],
                         preferred_element_type=jnp.float32)

  return pl.pallas_call(
      body,
      grid=(n // _BLK,),
      in_specs=[
          pl.BlockSpec((_BLK, d_in), lambda i: (i, 0)),
          pl.BlockSpec((d_in, d_out), lambda i: (0, 0)),
      ],
      out_specs=pl.BlockSpec((_BLK, d_out), lambda i: (i, 0)),
      out_shape=jax.ShapeDtypeStruct((n, d_out), jnp.float32),
  )(h, w)


def _tc_dis_scale(deg_t, mm):
  n, d = mm.shape
  nw = deg_t.shape[1]

  def body(deg_ref, mm_ref, dis_ref, y_ref):
    deg = jnp.sum(deg_ref[...], axis=1, keepdims=True) + 1.0
    dis = lax.rsqrt(deg)
    dis_ref[...] = dis
    y_ref[...] = mm_ref[...] * dis

  return pl.pallas_call(
      body,
      grid=(n // _BLK,),
      in_specs=[
          pl.BlockSpec((_BLK, nw), lambda i: (i, 0)),
          pl.BlockSpec((_BLK, d), lambda i: (i, 0)),
      ],
      out_specs=(
          pl.BlockSpec((_BLK, 1), lambda i: (i, 0)),
          pl.BlockSpec((_BLK, d), lambda i: (i, 0)),
      ),
      out_shape=(
          jax.ShapeDtypeStruct((n, 1), jnp.float32),
          jax.ShapeDtypeStruct((n, d), jnp.float32),
      ),
  )(deg_t, mm)


def _tc_layer(parts, y, dis, b, w_next):
  n, d = y.shape

  def body(p_ref, y_ref, d_ref, b_ref, w_ref, o_ref):
    t = (p_ref[0] + p_ref[1] + y_ref[...]) * d_ref[...] + b_ref[...]
    z = jnp.maximum(t, 0.0)
    o_ref[...] = (
        jnp.dot(z, w_ref[...], preferred_element_type=jnp.float32) * d_ref[...]
    )

  return pl.pallas_call(
      body,
      grid=(n // _BLK,),
      in_specs=[
          pl.BlockSpec((_NC, _BLK, d), lambda i: (0, i, 0)),
          pl.BlockSpec((_BLK, d), lambda i: (i, 0)),
          pl.BlockSpec((_BLK, 1), lambda i: (i, 0)),
          pl.BlockSpec((1, d), lambda i: (0, 0)),
          pl.BlockSpec((d, d), lambda i: (0, 0)),
      ],
      out_specs=pl.BlockSpec((_BLK, d), lambda i: (i, 0)),
      out_shape=jax.ShapeDtypeStruct((n, d), jnp.float32),
  )(parts, y, dis, b, w_next)


def _tc_final(parts, y, dis, b, batch_row, batch_col, gstats,
              a_mean, a_max, a_sum, a_st, mb1, m2, mb2, num_graphs):
  n, d = y.shape
  g = num_graphs
  mh = m2.shape[0]
  blk = 400
  nb = n // blk
  assert nb * blk == n

  def body(p_ref, y_ref, d_ref, b_ref, br_ref, bc_ref, gs_ref,
           am_ref, ax_ref, as_ref, ast_ref, mb1_ref, m2_ref, mb2_ref, o_ref,
           ssum_sc, smax_sc, cnt_sc):
    i = pl.program_id(0)

    @pl.when(i == 0)
    def _():
      ssum_sc[...] = jnp.zeros_like(ssum_sc)
      smax_sc[...] = jnp.full_like(smax_sc, -jnp.inf)
      cnt_sc[...] = jnp.zeros_like(cnt_sc)

    t = (p_ref[0] + p_ref[1] + y_ref[...]) * d_ref[...] + b_ref[...]
    h = jnp.maximum(t, 0.0)

    gid = lax.broadcasted_iota(jnp.int32, (g, 1), 0)
    oh = (br_ref[0] == gid).astype(jnp.float32)
    ssum_sc[...] += jnp.dot(oh, h, preferred_element_type=jnp.float32)
    cnt_sc[...] += jnp.sum(oh, axis=1, keepdims=True)

    bc = bc_ref[...]
    neg = jnp.float32(-jnp.inf)
    mx_rows = []
    for gg in range(g):
      m = jnp.where(bc == gg, h, neg)
      mx_rows.append(jnp.max(m, axis=0, keepdims=True))
    smax_sc[...] = jnp.maximum(smax_sc[...], jnp.concatenate(mx_rows, axis=0))

    @pl.when(i == nb - 1)
    def _():
      ssum = ssum_sc[...]
      mean = ssum / jnp.maximum(cnt_sc[...], 1.0)
      zpre = (
          jnp.dot(mean, am_ref[...], preferred_element_type=jnp.float32)
          + jnp.dot(smax_sc[...], ax_ref[...],
                    preferred_element_type=jnp.float32)
          + jnp.dot(ssum, as_ref[...], preferred_element_type=jnp.float32)
          + mb1_ref[...]
      )
      gs = gs_ref[...]
      for kk in range(gs.shape[1]):
        zpre = zpre + gs[:, kk:kk + 1] * ast_ref[kk:kk + 1, :]
      z = jnp.maximum(zpre, 0.0)
      o_ref[...] = (
          jnp.dot(z, m2_ref[...], preferred_element_type=jnp.float32)
          + mb2_ref[...]
      )

  return pl.pallas_call(
      body,
      grid=(nb,),
      in_specs=[
          pl.BlockSpec((_NC, blk, d), lambda i: (0, i, 0)),
          pl.BlockSpec((blk, d), lambda i: (i, 0)),
          pl.BlockSpec((blk, 1), lambda i: (i, 0)),
          pl.BlockSpec((1, d), lambda i: (0, 0)),
          pl.BlockSpec((1, 1, blk), lambda i: (i, 0, 0)),
          pl.BlockSpec((blk, 1), lambda i: (i, 0)),
          pl.BlockSpec((g, 3), lambda i: (0, 0)),
          pl.BlockSpec((d, mh), lambda i: (0, 0)),
          pl.BlockSpec((d, mh), lambda i: (0, 0)),
          pl.BlockSpec((d, mh), lambda i: (0, 0)),
          pl.BlockSpec((3, mh), lambda i: (0, 0)),
          pl.BlockSpec((1, mh), lambda i: (0, 0)),
          pl.BlockSpec((mh, 1), lambda i: (0, 0)),
          pl.BlockSpec((1, 1), lambda i: (0, 0)),
      ],
      out_specs=pl.BlockSpec((g, 1), lambda i: (0, 0)),
      out_shape=jax.ShapeDtypeStruct((g, 1), jnp.float32),
      scratch_shapes=[
          pltpu.VMEM((g, d), jnp.float32),
          pltpu.VMEM((g, d), jnp.float32),
          pltpu.VMEM((g, 1), jnp.float32),
      ],
  )(parts, y, dis, b, batch_row, batch_col, gstats,
    a_mean, a_max, a_sum, a_st, mb1, m2, mb2)


def kernel(x, edge_index, batch, graph_stats, W1, b1, W2, b2, W3, b3,
           M1, mb1, M2, mb2):
  n, d_in = x.shape
  h = W1.shape[1]
  g = graph_stats.shape[0]
  src = edge_index[0].astype(jnp.int32)
  dst = edge_index[1].astype(jnp.int32)

  n_pad = ((n + 16 * _LANES - 1) // (16 * _LANES)) * (16 * _LANES)
  deg_parts = _sc_degree(dst, n_pad)
  deg_t = deg_parts.reshape(_NC, n_pad).T

  mm = _tc_matmul(x, W1)
  dis, y = _tc_dis_scale(deg_t, mm)

  parts = _sc_scatter(y, src, dst)
  y = _tc_layer(parts, y, dis, b1.reshape(1, h), W2)
  parts = _sc_scatter(y, src, dst)
  y = _tc_layer(parts, y, dis, b2.reshape(1, h), W3)
  parts = _sc_scatter(y, src, dst)

  a_mean = M1[:h]
  a_max = M1[h:2 * h]
  a_sum = M1[2 * h:3 * h]
  a_st = M1[3 * h:]
  out = _tc_final(
      parts, y, dis, b3.reshape(1, h),
      batch.astype(jnp.int32).reshape(-1, 1, 400),
      batch.astype(jnp.int32).reshape(n, 1),
      graph_stats,
      a_mean, a_max, a_sum, a_st,
      mb1.reshape(1, -1), M2, mb2.reshape(1, 1), g,
  )
  return jnp.squeeze(out)


# speedup vs baseline: 31.3437x; 1.0036x over previous
import dataclasses
import functools

import jax
import jax.numpy as jnp
from jax import lax
from jax.experimental import pallas as pl
from jax.experimental.pallas import tpu as pltpu
from jax.experimental.pallas import tpu_sc as plsc

_NC = 2
_NS = 16
_LANES = 16


def _sc_params():
  cp = pltpu.CompilerParams()
  if "needs_layout_passes" in pltpu.CompilerParams.__dataclass_fields__:
    cp = dataclasses.replace(cp, needs_layout_passes=False)
  return cp


def _sc_degree(dst, n_pad):
  e = dst.shape[0]
  nw = _NC * _NS
  per = e // nw
  col = n_pad // _NS
  assert per * nw == e and per % _LANES == 0 and col % 128 == 0
  mesh = plsc.VectorSubcoreMesh(core_axis_name="c", subcore_axis_name="s")

  @functools.partial(
      pl.kernel,
      out_type=jax.ShapeDtypeStruct((_NC * n_pad,), jnp.float32),
      mesh=mesh,
      scratch_types=[
          pltpu.VMEM((n_pad,), jnp.float32),
          pltpu.VMEM((per,), jnp.int32),
          pltpu.VMEM((_NS, col), jnp.float32),
          pltpu.VMEM_SHARED((_NS, n_pad), jnp.float32),
      ],
      compiler_params=_sc_params(),
  )
  def k(dst_hbm, out_hbm, hist_v, idx_v, comb_v, sh):
    c = lax.axis_index("c")
    s = lax.axis_index("s")
    w = c * _NS + s

    @pl.loop(0, n_pad // _LANES)
    def _(i):
      hist_v[pl.ds(i * _LANES, _LANES)] = jnp.zeros((_LANES,), jnp.float32)

    pltpu.sync_copy(dst_hbm.at[pl.ds(w * per, per)], idx_v)
    ones = jnp.ones((_LANES,), jnp.float32)

    @pl.loop(0, per // _LANES, unroll=4)
    def _(i):
      idx = idx_v[pl.ds(i * _LANES, _LANES)]
      plsc.addupdate_scatter(hist_v, [idx], ones)

    pltpu.sync_copy(hist_v, sh.at[s])
    plsc.subcore_barrier()
    pltpu.sync_copy(sh.at[:, pl.ds(s * col, col)], comb_v)

    @pl.loop(0, col // _LANES, unroll=4)
    def _(j):
      acc = comb_v[0, pl.ds(j * _LANES, _LANES)]
      for r in range(1, _NS):
        acc = acc + comb_v[r, pl.ds(j * _LANES, _LANES)]
      comb_v[0, pl.ds(j * _LANES, _LANES)] = acc

    pltpu.sync_copy(comb_v.at[0],
                    out_hbm.at[pl.ds(c * n_pad + s * col, col)])

  return k(dst)


def _sc_scatter(y, src, dst):
  n, d = y.shape
  e = src.shape[0]
  nw = _NC * _NS
  per = e // nw
  k_ch = 80
  nch = per // k_ch
  assert per * nw == e and nch * k_ch == per and (nch - 5) % 3 == 0
  rpt = (n // _NS) // 8 * 8
  rem = n - rpt * _NS
  assert rem % 8 == 0
  zr = 48
  nz = rpt // zr
  assert zr * nz == rpt and rem <= zr
  mesh = plsc.VectorSubcoreMesh(core_axis_name="c", subcore_axis_name="s")

  @functools.partial(
      pl.kernel,
      out_type=jax.ShapeDtypeStruct((_NC, n, d), jnp.float32),
      mesh=mesh,
      scratch_types=[
          pltpu.VMEM((per,), jnp.int32),
          pltpu.VMEM((k_ch,), jnp.int32),
          pltpu.VMEM((k_ch,), jnp.int32),
          pltpu.VMEM((k_ch,), jnp.int32),
          pltpu.VMEM((k_ch, d), jnp.float32),
          pltpu.VMEM((k_ch, d), jnp.float32),
          pltpu.VMEM((k_ch, d), jnp.float32),
          pltpu.VMEM((zr, d), jnp.float32),
          pltpu.VMEM_SHARED((n, d), jnp.float32),
          pltpu.SemaphoreType.DMA,
          pltpu.SemaphoreType.DMA,
          pltpu.SemaphoreType.DMA,
          pltpu.SemaphoreType.DMA,
          pltpu.SemaphoreType.DMA,
          pltpu.SemaphoreType.DMA,
          pltpu.SemaphoreType.DMA,
          pltpu.SemaphoreType.DMA,
          pltpu.SemaphoreType.DMA,
          pltpu.SemaphoreType.DMA,
      ],
      compiler_params=_sc_params(),
  )
  def k(y_hbm, src_hbm, dst_hbm, out_hbm, src_v, dst_a, dst_b, dst_c,
        buf_a, buf_b, buf_c, zero_v, acc_sh,
        gsem_a, gsem_b, gsem_c, ssem_a, ssem_b, ssem_c,
        dsem_a, dsem_b, dsem_c, zsem):
    c = lax.axis_index("c")
    s = lax.axis_index("s")
    w = c * _NS + s
    bufs = (buf_a, buf_b, buf_c)
    dsts = (dst_a, dst_b, dst_c)
    gsems = (gsem_a, gsem_b, gsem_c)
    ssems = (ssem_a, ssem_b, ssem_c)
    dsems = (dsem_a, dsem_b, dsem_c)

    base0 = w * per
    pltpu.sync_copy(src_hbm.at[pl.ds(base0, per)], src_v)

    def g_start(i, p):
      pltpu.async_copy(y_hbm.at[src_v.at[pl.ds(i * k_ch, k_ch)]],
                       bufs[p], gsems[p])

    def g_wait(p):
      pltpu.make_async_copy(y_hbm.at[src_v.at[pl.ds(0, k_ch)]],
                            bufs[p], gsems[p]).wait()

    def d_start(i, p):
      pltpu.async_copy(dst_hbm.at[pl.ds(base0 + i * k_ch, k_ch)],
                       dsts[p], dsems[p])

    def d_wait(p):
      pltpu.make_async_copy(dst_hbm.at[pl.ds(base0, k_ch)],
                            dsts[p], dsems[p]).wait()

    def s_start(p):
      pltpu.async_copy(bufs[p], acc_sh.at[dsts[p]], ssems[p], add=True)

    def s_wait(p):
      pltpu.make_async_copy(bufs[p], acc_sh.at[dsts[p]], ssems[p]).wait()

    d_start(0, 0)
    g_start(0, 0)
    d_start(1, 1)
    g_start(1, 1)

    @pl.loop(0, zr)
    def _(i):
      for j in range(d // _LANES):
        zero_v[i, pl.ds(j * _LANES, _LANES)] = jnp.zeros((_LANES,), jnp.float32)

    for t in range(nz):
      pltpu.async_copy(zero_v, acc_sh.at[pl.ds(s * rpt + t * zr, zr)], zsem)
    for t in range(nz):
      pltpu.make_async_copy(zero_v, acc_sh.at[pl.ds(s * rpt, zr)], zsem).wait()

    @pl.when(s == 0)
    def _():
      pltpu.async_copy(zero_v.at[pl.ds(0, rem)],
                       acc_sh.at[pl.ds(_NS * rpt, rem)], zsem)
      pltpu.make_async_copy(zero_v.at[pl.ds(0, rem)],
                            acc_sh.at[pl.ds(_NS * rpt, rem)], zsem).wait()

    plsc.subcore_barrier()

    def step(i, p, first_round):
      g_wait(p)
      q = (p + 2) % 3
      if not (first_round and q == 2):
        s_wait(q)
      d_start(i + 2, q)
      g_start(i + 2, q)
      d_wait(p)
      s_start(p)

    step(0, 0, True)
    step(1, 1, True)
    step(2, 2, False)

    @pl.loop(0, (nch - 5) // 3)
    def _(j):
      base = 3 + 3 * j
      step(base + 0, 0, False)
      step(base + 1, 1, False)
      step(base + 2, 2, False)

    g_wait(0)
    d_wait(0)
    s_start(0)
    g_wait(1)
    d_wait(1)
    s_start(1)
    s_wait(0)
    s_wait(1)
    s_wait(2)

    plsc.subcore_barrier()
    pltpu.sync_copy(
        acc_sh.at[pl.ds(s * rpt, rpt)],
        out_hbm.at[c, pl.ds(s * rpt, rpt)],
    )

    @pl.when(s == 0)
    def _():
      pltpu.sync_copy(
          acc_sh.at[pl.ds(_NS * rpt, rem)],
          out_hbm.at[c, pl.ds(_NS * rpt, rem)],
      )

  return k(y, src, dst)


_BLK = 2000


def _tc_matmul(h, w):
  n, d_in = h.shape
  d_out = w.shape[1]

  def body(h_ref, w_ref, y_ref):
    y_ref[...] = jnp.dot(h_ref[...], w_ref[...],
                         preferred_element_type=jnp.float32)

  return pl.pallas_call(
      body,
      grid=(n // _BLK,),
      in_specs=[
          pl.BlockSpec((_BLK, d_in), lambda i: (i, 0)),
          pl.BlockSpec((d_in, d_out), lambda i: (0, 0)),
      ],
      out_specs=pl.BlockSpec((_BLK, d_out), lambda i: (i, 0)),
      out_shape=jax.ShapeDtypeStruct((n, d_out), jnp.float32),
  )(h, w)


def _tc_dis_scale(deg_t, mm):
  n, d = mm.shape
  nw = deg_t.shape[1]

  def body(deg_ref, mm_ref, dis_ref, y_ref):
    deg = jnp.sum(deg_ref[...], axis=1, keepdims=True) + 1.0
    dis = lax.rsqrt(deg)
    dis_ref[...] = dis
    y_ref[...] = mm_ref[...] * dis

  return pl.pallas_call(
      body,
      grid=(n // _BLK,),
      in_specs=[
          pl.BlockSpec((_BLK, nw), lambda i: (i, 0)),
          pl.BlockSpec((_BLK, d), lambda i: (i, 0)),
      ],
      out_specs=(
          pl.BlockSpec((_BLK, 1), lambda i: (i, 0)),
          pl.BlockSpec((_BLK, d), lambda i: (i, 0)),
      ),
      out_shape=(
          jax.ShapeDtypeStruct((n, 1), jnp.float32),
          jax.ShapeDtypeStruct((n, d), jnp.float32),
      ),
  )(deg_t, mm)


def _tc_layer(parts, y, dis, b, w_next):
  n, d = y.shape

  def body(p_ref, y_ref, d_ref, b_ref, w_ref, o_ref):
    t = (p_ref[0] + p_ref[1] + y_ref[...]) * d_ref[...] + b_ref[...]
    z = jnp.maximum(t, 0.0)
    o_ref[...] = (
        jnp.dot(z, w_ref[...], preferred_element_type=jnp.float32) * d_ref[...]
    )

  return pl.pallas_call(
      body,
      grid=(n // _BLK,),
      in_specs=[
          pl.BlockSpec((_NC, _BLK, d), lambda i: (0, i, 0)),
          pl.BlockSpec((_BLK, d), lambda i: (i, 0)),
          pl.BlockSpec((_BLK, 1), lambda i: (i, 0)),
          pl.BlockSpec((1, d), lambda i: (0, 0)),
          pl.BlockSpec((d, d), lambda i: (0, 0)),
      ],
      out_specs=pl.BlockSpec((_BLK, d), lambda i: (i, 0)),
      out_shape=jax.ShapeDtypeStruct((n, d), jnp.float32),
  )(parts, y, dis, b, w_next)


def _tc_final(parts, y, dis, b, batch_row, batch_col, gstats,
              a_mean, a_max, a_sum, a_st, mb1, m2, mb2, num_graphs):
  n, d = y.shape
  g = num_graphs
  mh = m2.shape[0]
  blk = 400
  nb = n // blk
  assert nb * blk == n

  def body(p_ref, y_ref, d_ref, b_ref, br_ref, bc_ref, gs_ref,
           am_ref, ax_ref, as_ref, ast_ref, mb1_ref, m2_ref, mb2_ref, o_ref,
           ssum_sc, smax_sc, cnt_sc):
    i = pl.program_id(0)

    @pl.when(i == 0)
    def _():
      ssum_sc[...] = jnp.zeros_like(ssum_sc)
      smax_sc[...] = jnp.full_like(smax_sc, -jnp.inf)
      cnt_sc[...] = jnp.zeros_like(cnt_sc)

    t = (p_ref[0] + p_ref[1] + y_ref[...]) * d_ref[...] + b_ref[...]
    h = jnp.maximum(t, 0.0)

    gid = lax.broadcasted_iota(jnp.int32, (g, 1), 0)
    oh = (br_ref[0] == gid).astype(jnp.float32)
    ssum_sc[...] += jnp.dot(oh, h, preferred_element_type=jnp.float32)
    cnt_sc[...] += jnp.sum(oh, axis=1, keepdims=True)

    bc = bc_ref[...]
    neg = jnp.float32(-jnp.inf)
    mx_rows = []
    for gg in range(g):
      m = jnp.where(bc == gg, h, neg)
      mx_rows.append(jnp.max(m, axis=0, keepdims=True))
    smax_sc[...] = jnp.maximum(smax_sc[...], jnp.concatenate(mx_rows, axis=0))

    @pl.when(i == nb - 1)
    def _():
      ssum = ssum_sc[...]
      mean = ssum / jnp.maximum(cnt_sc[...], 1.0)
      zpre = (
          jnp.dot(mean, am_ref[...], preferred_element_type=jnp.float32)
          + jnp.dot(smax_sc[...], ax_ref[...],
                    preferred_element_type=jnp.float32)
          + jnp.dot(ssum, as_ref[...], preferred_element_type=jnp.float32)
          + mb1_ref[...]
      )
      gs = gs_ref[...]
      for kk in range(gs.shape[1]):
        zpre = zpre + gs[:, kk:kk + 1] * ast_ref[kk:kk + 1, :]
      z = jnp.maximum(zpre, 0.0)
      o_ref[...] = (
          jnp.dot(z, m2_ref[...], preferred_element_type=jnp.float32)
          + mb2_ref[...]
      )

  return pl.pallas_call(
      body,
      grid=(nb,),
      in_specs=[
          pl.BlockSpec((_NC, blk, d), lambda i: (0, i, 0)),
          pl.BlockSpec((blk, d), lambda i: (i, 0)),
          pl.BlockSpec((blk, 1), lambda i: (i, 0)),
          pl.BlockSpec((1, d), lambda i: (0, 0)),
          pl.BlockSpec((1, 1, blk), lambda i: (i, 0, 0)),
          pl.BlockSpec((blk, 1), lambda i: (i, 0)),
          pl.BlockSpec((g, 3), lambda i: (0, 0)),
          pl.BlockSpec((d, mh), lambda i: (0, 0)),
          pl.BlockSpec((d, mh), lambda i: (0, 0)),
          pl.BlockSpec((d, mh), lambda i: (0, 0)),
          pl.BlockSpec((3, mh), lambda i: (0, 0)),
          pl.BlockSpec((1, mh), lambda i: (0, 0)),
          pl.BlockSpec((mh, 1), lambda i: (0, 0)),
          pl.BlockSpec((1, 1), lambda i: (0, 0)),
      ],
      out_specs=pl.BlockSpec((g, 1), lambda i: (0, 0)),
      out_shape=jax.ShapeDtypeStruct((g, 1), jnp.float32),
      scratch_shapes=[
          pltpu.VMEM((g, d), jnp.float32),
          pltpu.VMEM((g, d), jnp.float32),
          pltpu.VMEM((g, 1), jnp.float32),
      ],
  )(parts, y, dis, b, batch_row, batch_col, gstats,
    a_mean, a_max, a_sum, a_st, mb1, m2, mb2)


def kernel(x, edge_index, batch, graph_stats, W1, b1, W2, b2, W3, b3,
           M1, mb1, M2, mb2):
  n, d_in = x.shape
  h = W1.shape[1]
  g = graph_stats.shape[0]
  src = edge_index[0].astype(jnp.int32)
  dst = edge_index[1].astype(jnp.int32)

  n_pad = ((n + 16 * _LANES - 1) // (16 * _LANES)) * (16 * _LANES)
  deg_parts = _sc_degree(dst, n_pad)
  deg_t = deg_parts.reshape(_NC, n_pad).T

  mm = _tc_matmul(x, W1)
  dis, y = _tc_dis_scale(deg_t, mm)

  parts = _sc_scatter(y, src, dst)
  y = _tc_layer(parts, y, dis, b1.reshape(1, h), W2)
  parts = _sc_scatter(y, src, dst)
  y = _tc_layer(parts, y, dis, b2.reshape(1, h), W3)
  parts = _sc_scatter(y, src, dst)

  a_mean = M1[:h]
  a_max = M1[h:2 * h]
  a_sum = M1[2 * h:3 * h]
  a_st = M1[3 * h:]
  out = _tc_final(
      parts, y, dis, b3.reshape(1, h),
      batch.astype(jnp.int32).reshape(-1, 1, 400),
      batch.astype(jnp.int32).reshape(n, 1),
      graph_stats,
      a_mean, a_max, a_sum, a_st,
      mb1.reshape(1, -1), M2, mb2.reshape(1, 1), g,
  )
  return jnp.squeeze(out)
